# Initial kernel scaffold; baseline (speedup 1.0000x reference)
import jax
import jax.numpy as jnp
from jax.experimental import pallas as pl


def kernel(x, edge_attr, global_attr, params, edge_index):
    raise NotImplementedError("write your pallas kernel here")


# speedup vs baseline: 1.1012x; 1.1012x over previous
import functools

import jax
import jax.numpy as jnp
import numpy as np
from jax.experimental import pallas as pl
from jax.experimental.pallas import tpu as pltpu

LAT = 128
_EB = 2000
_NB = 2000
_F32 = jnp.float32


def _split_rows(W, dims):
    out, o = [], 0
    for d in dims:
        out.append(W[o:o + d])
        o += d
    return out


def _mm_body(has_base, na, *refs):
    a = refs[:na]
    w = refs[na:2 * na]
    acc = jnp.dot(a[0][...], w[0][...], preferred_element_type=_F32)
    for i in range(1, na):
        acc = acc + jnp.dot(a[i][...], w[i][...], preferred_element_type=_F32)
    if has_base:
        acc = acc + refs[2 * na][...]
    refs[-1][...] = acc


def _mm(as_, ws, base=None, block=_NB):
    R = as_[0].shape[0]
    K = ws[0].shape[1]
    na = len(as_)
    in_specs = (
        [pl.BlockSpec((block, a.shape[1]), lambda i: (i, 0)) for a in as_]
        + [pl.BlockSpec((w.shape[0], K), lambda i: (0, 0)) for w in ws]
    )
    args = list(as_) + list(ws)
    if base is not None:
        in_specs.append(pl.BlockSpec((block, K), lambda i: (i, 0)))
        args.append(base)
    return pl.pallas_call(
        functools.partial(_mm_body, base is not None, na),
        grid=(R // block,),
        in_specs=in_specs,
        out_specs=pl.BlockSpec((block, K), lambda i: (i, 0)),
        out_shape=jax.ShapeDtypeStruct((R, K), _F32),
    )(*args)


def _edge_body(ne, *refs):
    acc = refs[0][...] + refs[1][...] + refs[2 + 2 * ne][...]
    for i in range(ne):
        acc = acc + jnp.dot(refs[2 + i][...], refs[2 + ne + i][...],
                            preferred_element_type=_F32)
    h = jnp.maximum(acc, 0.0)
    refs[-1][...] = (jnp.dot(h, refs[3 + 2 * ne][...],
                             preferred_element_type=_F32) + refs[4 + 2 * ne][...])


def _edge_mlp(gs, gd, ea_parts, we_parts, cvec, w2, b2):
    E = gs.shape[0]
    ne = len(ea_parts)
    d_out = w2.shape[1]
    in_specs = (
        [pl.BlockSpec((_EB, LAT), lambda i: (i, 0)),
         pl.BlockSpec((_EB, LAT), lambda i: (i, 0))]
        + [pl.BlockSpec((_EB, ea.shape[1]), lambda i: (i, 0)) for ea in ea_parts]
        + [pl.BlockSpec((we.shape[0], LAT), lambda i: (0, 0)) for we in we_parts]
        + [pl.BlockSpec((1, LAT), lambda i: (0, 0)),
           pl.BlockSpec((LAT, d_out), lambda i: (0, 0)),
           pl.BlockSpec((1, d_out), lambda i: (0, 0))]
    )
    return pl.pallas_call(
        functools.partial(_edge_body, ne),
        grid=(E // _EB,),
        in_specs=in_specs,
        out_specs=pl.BlockSpec((_EB, d_out), lambda i: (i, 0)),
        out_shape=jax.ShapeDtypeStruct((E, d_out), _F32),
    )(gs, gd, *ea_parts, *we_parts, cvec, w2, b2)


def _node_body(hx, agg, va, cvec, v2, b2, out):
    h = jnp.maximum(hx[...] + jnp.dot(agg[...], va[...],
                                      preferred_element_type=_F32) + cvec[...], 0.0)
    out[...] = jnp.dot(h, v2[...], preferred_element_type=_F32) + b2[...]


def _node_mlp(hx, agg, va, cvec, v2, b2):
    N = hx.shape[0]
    da = agg.shape[1]
    d_out = v2.shape[1]
    return pl.pallas_call(
        _node_body,
        grid=(N // _NB,),
        in_specs=[
            pl.BlockSpec((_NB, LAT), lambda i: (i, 0)),
            pl.BlockSpec((_NB, da), lambda i: (i, 0)),
            pl.BlockSpec((da, LAT), lambda i: (0, 0)),
            pl.BlockSpec((1, LAT), lambda i: (0, 0)),
            pl.BlockSpec((LAT, d_out), lambda i: (0, 0)),
            pl.BlockSpec((1, d_out), lambda i: (0, 0)),
        ],
        out_specs=pl.BlockSpec((_NB, d_out), lambda i: (i, 0)),
        out_shape=jax.ShapeDtypeStruct((N, d_out), _F32),
    )(hx, agg, va, cvec, v2, b2)


def _global_body(nu, inv_n, *refs):
    m = jnp.sum(refs[0][...], axis=0, keepdims=True) * inv_n
    acc = jnp.dot(m, refs[1 + nu][...], preferred_element_type=_F32)
    for i in range(nu):
        acc = acc + jnp.dot(refs[1 + i][...], refs[2 + nu + i][...],
                            preferred_element_type=_F32)
    h = jnp.maximum(acc + refs[2 + 2 * nu][...], 0.0)
    refs[-1][...] = (jnp.dot(h, refs[3 + 2 * nu][...],
                             preferred_element_type=_F32) + refs[4 + 2 * nu][...])


def _global_mlp(xn, u_parts, gm, gu_parts, b1, g2, b2):
    N = xn.shape[0]
    nu = len(u_parts)
    d_out = g2.shape[1]
    in_specs = (
        [pl.BlockSpec((N, LAT), lambda: (0, 0))]
        + [pl.BlockSpec((1, u.shape[1]), lambda: (0, 0)) for u in u_parts]
        + [pl.BlockSpec((LAT, LAT), lambda: (0, 0))]
        + [pl.BlockSpec((w.shape[0], LAT), lambda: (0, 0)) for w in gu_parts]
        + [pl.BlockSpec((1, LAT), lambda: (0, 0)),
           pl.BlockSpec((LAT, d_out), lambda: (0, 0)),
           pl.BlockSpec((1, d_out), lambda: (0, 0))]
    )
    return pl.pallas_call(
        functools.partial(_global_body, nu, 1.0 / N),
        in_specs=in_specs,
        out_specs=pl.BlockSpec((1, d_out), lambda: (0, 0)),
        out_shape=jax.ShapeDtypeStruct((1, d_out), _F32),
    )(xn, *u_parts, gm, *gu_parts, b1, g2, b2)


def _cvec_body(nu, *refs):
    acc_e = refs[3 * nu][...]
    acc_n = refs[3 * nu + 1][...]
    for i in range(nu):
        u = refs[i][...]
        acc_e = acc_e + jnp.dot(u, refs[nu + i][...], preferred_element_type=_F32)
        acc_n = acc_n + jnp.dot(u, refs[2 * nu + i][...],
                                preferred_element_type=_F32)
    refs[-2][...] = acc_e
    refs[-1][...] = acc_n


def _gather_rows(table, idx):
    return jnp.take(table, idx, axis=0)


def _scatter_add(e, col, N):
    return jax.ops.segment_sum(e, col, num_segments=N)


def _weight_views(p, dims_x, dims_e, dims_u, de_out):
    pe, pn, pg = p["edge"], p["node"], p["global"]
    nx, nee, nuu = len(dims_x), len(dims_e), len(dims_u)
    parts = _split_rows(pe["w1"], dims_x + dims_x + dims_e + dims_u)
    W_src = parts[:nx]
    W_dst = parts[nx:2 * nx]
    W_ea = parts[2 * nx:2 * nx + nee]
    W_eu = parts[2 * nx + nee:]
    parts = _split_rows(pn["w1"], dims_x + [de_out] + dims_u)
    V_x = parts[:nx]
    V_a = parts[nx]
    V_u = parts[nx + 1:]
    parts = _split_rows(pg["w1"], [LAT] + dims_u)
    G_m = parts[0]
    G_u = parts[1:]
    W_prep = [jnp.concatenate([W_src[i], W_dst[i], V_x[i]], axis=1)
              for i in range(nx)]
    return dict(W_prep=W_prep, W_ea=W_ea, W_eu=W_eu, V_a=V_a, V_u=V_u,
                G_m=G_m, G_u=G_u, pe=pe, pn=pn, pg=pg)


def _apply(wv, x_parts, ea_parts, u_parts, row, col, N, E, prep_base=None):
    if prep_base is None:
        prep = _mm(x_parts, wv["W_prep"])
    else:
        prep = _mm([x_parts[-1]], [wv["W_prep"][-1]], base=prep_base)
    xs = prep[:, :LAT]
    xd = prep[:, LAT:2 * LAT]
    hx = prep[:, 2 * LAT:]
    cvec_e, cvec_n = _cvecs_call(u_parts, wv["W_eu"], wv["V_u"],
                                 wv["pe"]["b1"].reshape(1, LAT),
                                 wv["pn"]["b1"].reshape(1, LAT))
    gs = _gather_rows(xs, row)
    gd = _gather_rows(xd, col)
    e = _edge_mlp(gs, gd, ea_parts, wv["W_ea"], cvec_e,
                  wv["pe"]["w2"], wv["pe"]["b2"].reshape(1, -1))
    agg = _scatter_add(e, col, N)
    xn = _node_mlp(hx, agg, wv["V_a"], cvec_n,
                   wv["pn"]["w2"], wv["pn"]["b2"].reshape(1, -1))
    gu = _global_mlp(xn, u_parts, wv["G_m"], wv["G_u"],
                     wv["pg"]["b1"].reshape(1, LAT),
                     wv["pg"]["w2"], wv["pg"]["b2"].reshape(1, -1))
    return xn, e, gu


def _cvecs_call(u_parts, we_parts, wn_parts, b1e, b1n):
    nu = len(u_parts)
    return pl.pallas_call(
        functools.partial(_cvec_body, nu),
        in_specs=(
            [pl.BlockSpec((1, u.shape[1]), lambda: (0, 0)) for u in u_parts]
            + [pl.BlockSpec((w.shape[0], LAT), lambda: (0, 0)) for w in we_parts]
            + [pl.BlockSpec((w.shape[0], LAT), lambda: (0, 0)) for w in wn_parts]
            + [pl.BlockSpec((1, LAT), lambda: (0, 0)),
               pl.BlockSpec((1, LAT), lambda: (0, 0))]
        ),
        out_specs=[pl.BlockSpec((1, LAT), lambda: (0, 0)),
                   pl.BlockSpec((1, LAT), lambda: (0, 0))],
        out_shape=[jax.ShapeDtypeStruct((1, LAT), _F32),
                   jax.ShapeDtypeStruct((1, LAT), _F32)],
    )(*u_parts, *we_parts, *wn_parts, b1e, b1n)


def kernel(x, edge_attr, global_attr, params, edge_index):
    row, col = edge_index[0], edge_index[1]
    N = x.shape[0]
    E = edge_attr.shape[0]
    DN, DE, DG = x.shape[1], edge_attr.shape[1], global_attr.shape[1]

    wv_enc = _weight_views(params["encoder"], [DN], [DE], [DG], DE)
    wv_core = _weight_views(params["core"], [DN, DN], [DE, DE], [DG, DG], DE)
    wv_dec = _weight_views(params["decoder"], [DN], [DE], [DG], DE)

    x1, e1, u1 = _apply(wv_enc, [x], [edge_attr], [global_attr], row, col, N, E)
    x0, e0, u0 = x1, e1, u1

    prep_base0 = _mm([x0], [wv_core["W_prep"][0]])
    xc, ec, uc = x1, e1, u1
    for _ in range(5):
        xc, ec, uc = _apply(wv_core, [x0, xc], [e0, ec], [u0, uc],
                            row, col, N, E, prep_base=prep_base0)

    return _apply(wv_dec, [xc], [ec], [uc], row, col, N, E)


# speedup vs baseline: 3.9364x; 3.5747x over previous
import functools

import jax
import jax.numpy as jnp
import numpy as np
from jax import lax
from jax.experimental import pallas as pl
from jax.experimental.pallas import tpu as pltpu
from jax.experimental.pallas import tpu_sc as plsc

_NC = 2
_NS = 16
_NW = _NC * _NS

LAT = 128
_EB = 2000
_NB = 2000
_F32 = jnp.float32


def _split_rows(W, dims):
    out, o = [], 0
    for d in dims:
        out.append(W[o:o + d])
        o += d
    return out


def _mm_body(has_base, na, *refs):
    a = refs[:na]
    w = refs[na:2 * na]
    acc = jnp.dot(a[0][...], w[0][...], preferred_element_type=_F32)
    for i in range(1, na):
        acc = acc + jnp.dot(a[i][...], w[i][...], preferred_element_type=_F32)
    if has_base:
        acc = acc + refs[2 * na][...]
    refs[-1][...] = acc


def _mm(as_, ws, base=None, block=_NB):
    R = as_[0].shape[0]
    K = ws[0].shape[1]
    na = len(as_)
    in_specs = (
        [pl.BlockSpec((block, a.shape[1]), lambda i: (i, 0)) for a in as_]
        + [pl.BlockSpec((w.shape[0], K), lambda i: (0, 0)) for w in ws]
    )
    args = list(as_) + list(ws)
    if base is not None:
        in_specs.append(pl.BlockSpec((block, K), lambda i: (i, 0)))
        args.append(base)
    return pl.pallas_call(
        functools.partial(_mm_body, base is not None, na),
        grid=(R // block,),
        in_specs=in_specs,
        out_specs=pl.BlockSpec((block, K), lambda i: (i, 0)),
        out_shape=jax.ShapeDtypeStruct((R, K), _F32),
    )(*args)


def _edge_body(ne, *refs):
    acc = refs[0][...] + refs[1][...] + refs[2 + 2 * ne][...]
    for i in range(ne):
        acc = acc + jnp.dot(refs[2 + i][...], refs[2 + ne + i][...],
                            preferred_element_type=_F32)
    h = jnp.maximum(acc, 0.0)
    refs[-1][...] = (jnp.dot(h, refs[3 + 2 * ne][...],
                             preferred_element_type=_F32) + refs[4 + 2 * ne][...])


def _edge_mlp(gs, gd, ea_parts, we_parts, cvec, w2, b2):
    E = gs.shape[0]
    ne = len(ea_parts)
    d_out = w2.shape[1]
    in_specs = (
        [pl.BlockSpec((_EB, LAT), lambda i: (i, 0)),
         pl.BlockSpec((_EB, LAT), lambda i: (i, 0))]
        + [pl.BlockSpec((_EB, ea.shape[1]), lambda i: (i, 0)) for ea in ea_parts]
        + [pl.BlockSpec((we.shape[0], LAT), lambda i: (0, 0)) for we in we_parts]
        + [pl.BlockSpec((1, LAT), lambda i: (0, 0)),
           pl.BlockSpec((LAT, d_out), lambda i: (0, 0)),
           pl.BlockSpec((1, d_out), lambda i: (0, 0))]
    )
    return pl.pallas_call(
        functools.partial(_edge_body, ne),
        grid=(E // _EB,),
        in_specs=in_specs,
        out_specs=pl.BlockSpec((_EB, d_out), lambda i: (i, 0)),
        out_shape=jax.ShapeDtypeStruct((E, d_out), _F32),
    )(gs, gd, *ea_parts, *we_parts, cvec, w2, b2)


def _node_body(hx, aggp, va, cvec, v2, b2, out):
    agg = aggp[0] + aggp[1]
    h = jnp.maximum(hx[...] + jnp.dot(agg, va[...],
                                      preferred_element_type=_F32) + cvec[...], 0.0)
    out[...] = jnp.dot(h, v2[...], preferred_element_type=_F32) + b2[...]


def _node_mlp(hx, aggp, va, cvec, v2, b2):
    N = hx.shape[0]
    da = aggp.shape[2]
    d_out = v2.shape[1]
    return pl.pallas_call(
        _node_body,
        grid=(N // _NB,),
        in_specs=[
            pl.BlockSpec((_NB, LAT), lambda i: (i, 0)),
            pl.BlockSpec((_NC, _NB, da), lambda i: (0, i, 0)),
            pl.BlockSpec((da, LAT), lambda i: (0, 0)),
            pl.BlockSpec((1, LAT), lambda i: (0, 0)),
            pl.BlockSpec((LAT, d_out), lambda i: (0, 0)),
            pl.BlockSpec((1, d_out), lambda i: (0, 0)),
        ],
        out_specs=pl.BlockSpec((_NB, d_out), lambda i: (i, 0)),
        out_shape=jax.ShapeDtypeStruct((N, d_out), _F32),
    )(hx, aggp, va, cvec, v2, b2)


def _global_body(nu, inv_n, *refs):
    m = jnp.sum(refs[0][...], axis=0, keepdims=True) * inv_n
    acc = jnp.dot(m, refs[1 + nu][...], preferred_element_type=_F32)
    for i in range(nu):
        acc = acc + jnp.dot(refs[1 + i][...], refs[2 + nu + i][...],
                            preferred_element_type=_F32)
    h = jnp.maximum(acc + refs[2 + 2 * nu][...], 0.0)
    refs[-1][...] = (jnp.dot(h, refs[3 + 2 * nu][...],
                             preferred_element_type=_F32) + refs[4 + 2 * nu][...])


def _global_mlp(xn, u_parts, gm, gu_parts, b1, g2, b2):
    N = xn.shape[0]
    nu = len(u_parts)
    d_out = g2.shape[1]
    in_specs = (
        [pl.BlockSpec((N, LAT), lambda: (0, 0))]
        + [pl.BlockSpec((1, u.shape[1]), lambda: (0, 0)) for u in u_parts]
        + [pl.BlockSpec((LAT, LAT), lambda: (0, 0))]
        + [pl.BlockSpec((w.shape[0], LAT), lambda: (0, 0)) for w in gu_parts]
        + [pl.BlockSpec((1, LAT), lambda: (0, 0)),
           pl.BlockSpec((LAT, d_out), lambda: (0, 0)),
           pl.BlockSpec((1, d_out), lambda: (0, 0))]
    )
    return pl.pallas_call(
        functools.partial(_global_body, nu, 1.0 / N),
        in_specs=in_specs,
        out_specs=pl.BlockSpec((1, d_out), lambda: (0, 0)),
        out_shape=jax.ShapeDtypeStruct((1, d_out), _F32),
    )(xn, *u_parts, gm, *gu_parts, b1, g2, b2)


def _cvec_body(nu, *refs):
    acc_e = refs[3 * nu][...]
    acc_n = refs[3 * nu + 1][...]
    for i in range(nu):
        u = refs[i][...]
        acc_e = acc_e + jnp.dot(u, refs[nu + i][...], preferred_element_type=_F32)
        acc_n = acc_n + jnp.dot(u, refs[2 * nu + i][...],
                                preferred_element_type=_F32)
    refs[-2][...] = acc_e
    refs[-1][...] = acc_n


_GW = 400


def _gather_body(nwin, xs_hbm, xd_hbm, ridx_hbm, cidx_hbm, gs_hbm, gd_hbm,
                 idx_a, idx_b, buf_a, buf_b, sem_a, sem_b):
    wid = lax.axis_index("s") * _NC + lax.axis_index("c")

    def step(j, carry):
        r = wid * nwin + j
        pltpu.sync_copy(ridx_hbm.at[r], idx_a)
        pltpu.sync_copy(cidx_hbm.at[r], idx_b)
        cp_a = pltpu.async_copy(xs_hbm.at[idx_a], buf_a, sem_a)
        cp_b = pltpu.async_copy(xd_hbm.at[idx_b], buf_b, sem_b)
        cp_a.wait()
        cp_b.wait()
        wr_a = pltpu.async_copy(buf_a, gs_hbm.at[pl.ds(r * _GW, _GW)], sem_a)
        wr_b = pltpu.async_copy(buf_b, gd_hbm.at[pl.ds(r * _GW, _GW)], sem_b)
        wr_a.wait()
        wr_b.wait()
        return carry

    lax.fori_loop(0, nwin, step, 0)


def _sc_gather2(xs, xd, ridx2, cidx2, E):
    nwin = ridx2.shape[0] // _NW
    mesh = plsc.VectorSubcoreMesh(core_axis_name="c", subcore_axis_name="s",
                                  num_cores=_NC, num_subcores=_NS)
    fn = pl.kernel(
        functools.partial(_gather_body, nwin),
        out_type=[jax.ShapeDtypeStruct((E, LAT), _F32),
                  jax.ShapeDtypeStruct((E, LAT), _F32)],
        mesh=mesh,
        scratch_types=[
            pltpu.VMEM((_GW,), jnp.int32),
            pltpu.VMEM((_GW,), jnp.int32),
            pltpu.VMEM((_GW, LAT), _F32),
            pltpu.VMEM((_GW, LAT), _F32),
            pltpu.SemaphoreType.DMA,
            pltpu.SemaphoreType.DMA,
        ],
    )
    return fn(xs, xd, ridx2, cidx2)


def _scatter_body(nwin, nchunk, e_hbm, cidx_hbm, zero_hbm, out_hbm,
                  idx_v, upd_v, obuf, acc):
    cid = lax.axis_index("c")
    sid = lax.axis_index("s")
    wid = sid * _NC + cid
    nrows = acc.shape[0]

    @pl.when(sid * nchunk < nrows)
    def _():
        pltpu.sync_copy(zero_hbm.at[pl.ds(sid * nchunk, nchunk)], obuf)
        pltpu.sync_copy(obuf, acc.at[pl.ds(sid * nchunk, nchunk)])

    plsc.subcore_barrier()

    def step(j, carry):
        r = wid * nwin + j
        pltpu.sync_copy(cidx_hbm.at[r], idx_v)
        pltpu.sync_copy(e_hbm.at[pl.ds(r * _GW, _GW)], upd_v)
        pltpu.sync_copy(upd_v, acc.at[idx_v], add=True)
        return carry

    lax.fori_loop(0, nwin, step, 0)
    plsc.subcore_barrier()

    @pl.when(sid * nchunk < nrows)
    def _():
        pltpu.sync_copy(acc.at[pl.ds(sid * nchunk, nchunk)], obuf)
        pltpu.sync_copy(obuf, out_hbm.at[cid, pl.ds(sid * nchunk, nchunk)])


def _sc_scatter_add(e, cidx2, N):
    E, de = e.shape
    nwin = cidx2.shape[0] // _NW
    nchunk = 1000
    mesh = plsc.VectorSubcoreMesh(core_axis_name="c", subcore_axis_name="s",
                                  num_cores=_NC, num_subcores=_NS)
    fn = pl.kernel(
        functools.partial(_scatter_body, nwin, nchunk),
        out_type=jax.ShapeDtypeStruct((_NC, N, de), _F32),
        mesh=mesh,
        compiler_params=pltpu.CompilerParams(use_tc_tiling_on_sc=False),
        scratch_types=[
            pltpu.VMEM((_GW,), jnp.int32),
            pltpu.VMEM((_GW, de), _F32),
            pltpu.VMEM((1000, de), _F32),
            pltpu.VMEM_SHARED((N, de), _F32),
        ],
    )
    return fn(e, cidx2, jnp.zeros((N, de), _F32))


def _weight_views(p, dims_x, dims_e, dims_u, de_out):
    pe, pn, pg = p["edge"], p["node"], p["global"]
    nx, nee, nuu = len(dims_x), len(dims_e), len(dims_u)
    parts = _split_rows(pe["w1"], dims_x + dims_x + dims_e + dims_u)
    W_src = parts[:nx]
    W_dst = parts[nx:2 * nx]
    W_ea = parts[2 * nx:2 * nx + nee]
    W_eu = parts[2 * nx + nee:]
    parts = _split_rows(pn["w1"], dims_x + [de_out] + dims_u)
    V_x = parts[:nx]
    V_a = parts[nx]
    V_u = parts[nx + 1:]
    parts = _split_rows(pg["w1"], [LAT] + dims_u)
    G_m = parts[0]
    G_u = parts[1:]
    W_prep = [jnp.concatenate([W_src[i], W_dst[i], V_x[i]], axis=1)
              for i in range(nx)]
    return dict(W_prep=W_prep, W_ea=W_ea, W_eu=W_eu, V_a=V_a, V_u=V_u,
                G_m=G_m, G_u=G_u, pe=pe, pn=pn, pg=pg)


def _apply(wv, x_parts, ea_parts, u_parts, ridx2, cidx2, N, E, prep_base=None):
    if prep_base is None:
        prep = _mm(x_parts, wv["W_prep"])
    else:
        prep = _mm([x_parts[-1]], [wv["W_prep"][-1]], base=prep_base)
    xs = prep[:, :LAT]
    xd = prep[:, LAT:2 * LAT]
    hx = prep[:, 2 * LAT:]
    cvec_e, cvec_n = _cvecs_call(u_parts, wv["W_eu"], wv["V_u"],
                                 wv["pe"]["b1"].reshape(1, LAT),
                                 wv["pn"]["b1"].reshape(1, LAT))
    gs, gd = _sc_gather2(xs, xd, ridx2, cidx2, E)
    e = _edge_mlp(gs, gd, ea_parts, wv["W_ea"], cvec_e,
                  wv["pe"]["w2"], wv["pe"]["b2"].reshape(1, -1))
    aggp = _sc_scatter_add(e, cidx2, N)
    xn = _node_mlp(hx, aggp, wv["V_a"], cvec_n,
                   wv["pn"]["w2"], wv["pn"]["b2"].reshape(1, -1))
    gu = _global_mlp(xn, u_parts, wv["G_m"], wv["G_u"],
                     wv["pg"]["b1"].reshape(1, LAT),
                     wv["pg"]["w2"], wv["pg"]["b2"].reshape(1, -1))
    return xn, e, gu


def _cvecs_call(u_parts, we_parts, wn_parts, b1e, b1n):
    nu = len(u_parts)
    return pl.pallas_call(
        functools.partial(_cvec_body, nu),
        in_specs=(
            [pl.BlockSpec((1, u.shape[1]), lambda: (0, 0)) for u in u_parts]
            + [pl.BlockSpec((w.shape[0], LAT), lambda: (0, 0)) for w in we_parts]
            + [pl.BlockSpec((w.shape[0], LAT), lambda: (0, 0)) for w in wn_parts]
            + [pl.BlockSpec((1, LAT), lambda: (0, 0)),
               pl.BlockSpec((1, LAT), lambda: (0, 0))]
        ),
        out_specs=[pl.BlockSpec((1, LAT), lambda: (0, 0)),
                   pl.BlockSpec((1, LAT), lambda: (0, 0))],
        out_shape=[jax.ShapeDtypeStruct((1, LAT), _F32),
                   jax.ShapeDtypeStruct((1, LAT), _F32)],
    )(*u_parts, *we_parts, *wn_parts, b1e, b1n)


def kernel(x, edge_attr, global_attr, params, edge_index):
    row, col = edge_index[0], edge_index[1]
    N = x.shape[0]
    E = edge_attr.shape[0]
    DN, DE, DG = x.shape[1], edge_attr.shape[1], global_attr.shape[1]

    wv_enc = _weight_views(params["encoder"], [DN], [DE], [DG], DE)
    wv_core = _weight_views(params["core"], [DN, DN], [DE, DE], [DG, DG], DE)
    wv_dec = _weight_views(params["decoder"], [DN], [DE], [DG], DE)

    ridx2 = row.reshape(-1, _GW)
    cidx2 = col.reshape(-1, _GW)

    x1, e1, u1 = _apply(wv_enc, [x], [edge_attr], [global_attr],
                        ridx2, cidx2, N, E)
    x0, e0, u0 = x1, e1, u1

    prep_base0 = _mm([x0], [wv_core["W_prep"][0]])
    xc, ec, uc = x1, e1, u1
    for _ in range(5):
        xc, ec, uc = _apply(wv_core, [x0, xc], [e0, ec], [u0, uc],
                            ridx2, cidx2, N, E, prep_base=prep_base0)

    return _apply(wv_dec, [xc], [ec], [uc], ridx2, cidx2, N, E)


# speedup vs baseline: 4.2228x; 1.0728x over previous
import functools

import jax
import jax.numpy as jnp
import numpy as np
from jax import lax
from jax.experimental import pallas as pl
from jax.experimental.pallas import tpu as pltpu
from jax.experimental.pallas import tpu_sc as plsc

_NC = 2
_NS = 16
_NW = _NC * _NS

LAT = 128
_EB = 2000
_NB = 2000
_F32 = jnp.float32


def _split_rows(W, dims):
    out, o = [], 0
    for d in dims:
        out.append(W[o:o + d])
        o += d
    return out


def _mm_body(has_base, na, *refs):
    a = refs[:na]
    w = refs[na:2 * na]
    acc = jnp.dot(a[0][...], w[0][...], preferred_element_type=_F32)
    for i in range(1, na):
        acc = acc + jnp.dot(a[i][...], w[i][...], preferred_element_type=_F32)
    if has_base:
        acc = acc + refs[2 * na][...]
    refs[-1][...] = acc


def _mm(as_, ws, base=None, block=_NB):
    R = as_[0].shape[0]
    K = ws[0].shape[1]
    na = len(as_)
    in_specs = (
        [pl.BlockSpec((block, a.shape[1]), lambda i: (i, 0)) for a in as_]
        + [pl.BlockSpec((w.shape[0], K), lambda i: (0, 0)) for w in ws]
    )
    args = list(as_) + list(ws)
    if base is not None:
        in_specs.append(pl.BlockSpec((block, K), lambda i: (i, 0)))
        args.append(base)
    return pl.pallas_call(
        functools.partial(_mm_body, base is not None, na),
        grid=(R // block,),
        in_specs=in_specs,
        out_specs=pl.BlockSpec((block, K), lambda i: (i, 0)),
        out_shape=jax.ShapeDtypeStruct((R, K), _F32),
    )(*args)


def _edge_body(ne, *refs):
    acc = refs[0][...] + refs[1][...] + refs[2 + 2 * ne][...]
    for i in range(ne):
        acc = acc + jnp.dot(refs[2 + i][...], refs[2 + ne + i][...],
                            preferred_element_type=_F32)
    h = jnp.maximum(acc, 0.0)
    refs[-1][...] = (jnp.dot(h, refs[3 + 2 * ne][...],
                             preferred_element_type=_F32) + refs[4 + 2 * ne][...])


def _edge_mlp(gs, gd, ea_parts, we_parts, cvec, w2, b2):
    E = gs.shape[0]
    ne = len(ea_parts)
    d_out = w2.shape[1]
    in_specs = (
        [pl.BlockSpec((_EB, LAT), lambda i: (i, 0)),
         pl.BlockSpec((_EB, LAT), lambda i: (i, 0))]
        + [pl.BlockSpec((_EB, ea.shape[1]), lambda i: (i, 0)) for ea in ea_parts]
        + [pl.BlockSpec((we.shape[0], LAT), lambda i: (0, 0)) for we in we_parts]
        + [pl.BlockSpec((1, LAT), lambda i: (0, 0)),
           pl.BlockSpec((LAT, d_out), lambda i: (0, 0)),
           pl.BlockSpec((1, d_out), lambda i: (0, 0))]
    )
    return pl.pallas_call(
        functools.partial(_edge_body, ne),
        grid=(E // _EB,),
        in_specs=in_specs,
        out_specs=pl.BlockSpec((_EB, d_out), lambda i: (i, 0)),
        out_shape=jax.ShapeDtypeStruct((E, d_out), _F32),
    )(gs, gd, *ea_parts, *we_parts, cvec, w2, b2)


def _node_body(hx, aggp, va, cvec, v2, b2, out):
    agg = aggp[0] + aggp[1]
    h = jnp.maximum(hx[...] + jnp.dot(agg, va[...],
                                      preferred_element_type=_F32) + cvec[...], 0.0)
    out[...] = jnp.dot(h, v2[...], preferred_element_type=_F32) + b2[...]


def _node_mlp(hx, aggp, va, cvec, v2, b2):
    N = hx.shape[0]
    da = aggp.shape[2]
    d_out = v2.shape[1]
    return pl.pallas_call(
        _node_body,
        grid=(N // _NB,),
        in_specs=[
            pl.BlockSpec((_NB, LAT), lambda i: (i, 0)),
            pl.BlockSpec((_NC, _NB, da), lambda i: (0, i, 0)),
            pl.BlockSpec((da, LAT), lambda i: (0, 0)),
            pl.BlockSpec((1, LAT), lambda i: (0, 0)),
            pl.BlockSpec((LAT, d_out), lambda i: (0, 0)),
            pl.BlockSpec((1, d_out), lambda i: (0, 0)),
        ],
        out_specs=pl.BlockSpec((_NB, d_out), lambda i: (i, 0)),
        out_shape=jax.ShapeDtypeStruct((N, d_out), _F32),
    )(hx, aggp, va, cvec, v2, b2)


def _global_body(nu, inv_n, *refs):
    m = jnp.sum(refs[0][...], axis=0, keepdims=True) * inv_n
    acc = jnp.dot(m, refs[1 + nu][...], preferred_element_type=_F32)
    for i in range(nu):
        acc = acc + jnp.dot(refs[1 + i][...], refs[2 + nu + i][...],
                            preferred_element_type=_F32)
    h = jnp.maximum(acc + refs[2 + 2 * nu][...], 0.0)
    refs[-1][...] = (jnp.dot(h, refs[3 + 2 * nu][...],
                             preferred_element_type=_F32) + refs[4 + 2 * nu][...])


def _global_mlp(xn, u_parts, gm, gu_parts, b1, g2, b2):
    N = xn.shape[0]
    nu = len(u_parts)
    d_out = g2.shape[1]
    in_specs = (
        [pl.BlockSpec((N, LAT), lambda: (0, 0))]
        + [pl.BlockSpec((1, u.shape[1]), lambda: (0, 0)) for u in u_parts]
        + [pl.BlockSpec((LAT, LAT), lambda: (0, 0))]
        + [pl.BlockSpec((w.shape[0], LAT), lambda: (0, 0)) for w in gu_parts]
        + [pl.BlockSpec((1, LAT), lambda: (0, 0)),
           pl.BlockSpec((LAT, d_out), lambda: (0, 0)),
           pl.BlockSpec((1, d_out), lambda: (0, 0))]
    )
    return pl.pallas_call(
        functools.partial(_global_body, nu, 1.0 / N),
        in_specs=in_specs,
        out_specs=pl.BlockSpec((1, d_out), lambda: (0, 0)),
        out_shape=jax.ShapeDtypeStruct((1, d_out), _F32),
    )(xn, *u_parts, gm, *gu_parts, b1, g2, b2)


def _cvec_body(nu, *refs):
    acc_e = refs[3 * nu][...]
    acc_n = refs[3 * nu + 1][...]
    for i in range(nu):
        u = refs[i][...]
        acc_e = acc_e + jnp.dot(u, refs[nu + i][...], preferred_element_type=_F32)
        acc_n = acc_n + jnp.dot(u, refs[2 * nu + i][...],
                                preferred_element_type=_F32)
    refs[-2][...] = acc_e
    refs[-1][...] = acc_n


_GWG = 200
_GWS = 2000


def _gather_body(nwin, xs_hbm, xd_hbm, idx_hbm, gs_hbm, gd_hbm,
                 ix0, ix1, a0, b0, a1, b1, si0, si1, sg0, sg1, sw0, sw1):
    wid = lax.axis_index("s") * _NC + lax.axis_index("c")
    base = wid * nwin
    ix = (ix0, ix1)
    ab = ((a0, b0), (a1, b1))
    si = (si0, si1)
    sg = (sg0, sg1)
    sw = (sw0, sw1)

    def prefetch(w, slot):
        pltpu.async_copy(idx_hbm.at[base + w], ix[slot], si[slot])

    def window(w, slot, first):
        a, b = ab[slot]
        if not first:
            pltpu.make_async_copy(a, gs_hbm.at[pl.ds(0, _GWG)], sw[slot]).wait()
            pltpu.make_async_copy(b, gd_hbm.at[pl.ds(0, _GWG)], sw[slot]).wait()
        pltpu.make_async_copy(idx_hbm.at[base], ix[slot], si[slot]).wait()
        cpa = pltpu.async_copy(xs_hbm.at[ix[slot].at[pl.ds(0, _GWG)]], a, sg[slot])
        cpb = pltpu.async_copy(xd_hbm.at[ix[slot].at[pl.ds(_GWG, _GWG)]], b, sg[slot])
        cpa.wait()
        cpb.wait()
        if isinstance(w, int) and w + 2 < nwin:
            prefetch(w + 2, slot)
        elif not isinstance(w, int):
            @pl.when(w + 2 < nwin)
            def _():
                prefetch(w + 2, slot)
        pltpu.async_copy(a, gs_hbm.at[pl.ds((base + w) * _GWG, _GWG)], sw[slot])
        pltpu.async_copy(b, gd_hbm.at[pl.ds((base + w) * _GWG, _GWG)], sw[slot])

    prefetch(0, 0)
    prefetch(1, 1)
    window(0, 0, True)
    window(1, 1, True)

    def step(k, carry):
        window(2 * k, 0, False)
        window(2 * k + 1, 1, False)
        return carry

    lax.fori_loop(1, nwin // 2, step, 0)
    for slot in (0, 1):
        pltpu.make_async_copy(ab[slot][0], gs_hbm.at[pl.ds(0, _GWG)], sw[slot]).wait()
        pltpu.make_async_copy(ab[slot][1], gd_hbm.at[pl.ds(0, _GWG)], sw[slot]).wait()


def _sc_gather2(xs, xd, idx2, E):
    nwin = idx2.shape[0] // _NW
    mesh = plsc.VectorSubcoreMesh(core_axis_name="c", subcore_axis_name="s",
                                  num_cores=_NC, num_subcores=_NS)
    fn = pl.kernel(
        functools.partial(_gather_body, nwin),
        out_type=[jax.ShapeDtypeStruct((E, LAT), _F32),
                  jax.ShapeDtypeStruct((E, LAT), _F32)],
        mesh=mesh,
        scratch_types=[
            pltpu.VMEM((2 * _GWG,), jnp.int32),
            pltpu.VMEM((2 * _GWG,), jnp.int32),
            pltpu.VMEM((_GWG, LAT), _F32),
            pltpu.VMEM((_GWG, LAT), _F32),
            pltpu.VMEM((_GWG, LAT), _F32),
            pltpu.VMEM((_GWG, LAT), _F32),
            pltpu.SemaphoreType.DMA,
            pltpu.SemaphoreType.DMA,
            pltpu.SemaphoreType.DMA,
            pltpu.SemaphoreType.DMA,
            pltpu.SemaphoreType.DMA,
            pltpu.SemaphoreType.DMA,
        ],
    )
    return fn(xs, xd, idx2)


def _scatter_body(nwin, nchunk, e_hbm, cidx_hbm, zero_hbm, out_hbm,
                  idx_v, upd_v, obuf, acc):
    cid = lax.axis_index("c")
    sid = lax.axis_index("s")
    wid = sid * _NC + cid
    nrows = acc.shape[0]

    @pl.when(sid * nchunk < nrows)
    def _():
        pltpu.sync_copy(zero_hbm.at[pl.ds(sid * nchunk, nchunk)], obuf)
        pltpu.sync_copy(obuf, acc.at[pl.ds(sid * nchunk, nchunk)])

    plsc.subcore_barrier()

    def step(j, carry):
        r = wid * nwin + j
        pltpu.sync_copy(cidx_hbm.at[r], idx_v)
        pltpu.sync_copy(e_hbm.at[pl.ds(r * _GWS, _GWS)], upd_v)
        pltpu.sync_copy(upd_v, acc.at[idx_v], add=True)
        return carry

    lax.fori_loop(0, nwin, step, 0)
    plsc.subcore_barrier()

    @pl.when(sid * nchunk < nrows)
    def _():
        pltpu.sync_copy(acc.at[pl.ds(sid * nchunk, nchunk)], obuf)
        pltpu.sync_copy(obuf, out_hbm.at[cid, pl.ds(sid * nchunk, nchunk)])


def _sc_scatter_add(e, cidx2, N):
    E, de = e.shape
    nwin = cidx2.shape[0] // _NW
    nchunk = 1000
    mesh = plsc.VectorSubcoreMesh(core_axis_name="c", subcore_axis_name="s",
                                  num_cores=_NC, num_subcores=_NS)
    fn = pl.kernel(
        functools.partial(_scatter_body, nwin, nchunk),
        out_type=jax.ShapeDtypeStruct((_NC, N, de), _F32),
        mesh=mesh,
        compiler_params=pltpu.CompilerParams(use_tc_tiling_on_sc=False),
        scratch_types=[
            pltpu.VMEM((_GWS,), jnp.int32),
            pltpu.VMEM((_GWS, de), _F32),
            pltpu.VMEM((1000, de), _F32),
            pltpu.VMEM_SHARED((N, de), _F32),
        ],
    )
    return fn(e, cidx2, jnp.zeros((N, de), _F32))


def _weight_views(p, dims_x, dims_e, dims_u, de_out):
    pe, pn, pg = p["edge"], p["node"], p["global"]
    nx, nee, nuu = len(dims_x), len(dims_e), len(dims_u)
    parts = _split_rows(pe["w1"], dims_x + dims_x + dims_e + dims_u)
    W_src = parts[:nx]
    W_dst = parts[nx:2 * nx]
    W_ea = parts[2 * nx:2 * nx + nee]
    W_eu = parts[2 * nx + nee:]
    parts = _split_rows(pn["w1"], dims_x + [de_out] + dims_u)
    V_x = parts[:nx]
    V_a = parts[nx]
    V_u = parts[nx + 1:]
    parts = _split_rows(pg["w1"], [LAT] + dims_u)
    G_m = parts[0]
    G_u = parts[1:]
    W_prep = [jnp.concatenate([W_src[i], W_dst[i], V_x[i]], axis=1)
              for i in range(nx)]
    return dict(W_prep=W_prep, W_ea=W_ea, W_eu=W_eu, V_a=V_a, V_u=V_u,
                G_m=G_m, G_u=G_u, pe=pe, pn=pn, pg=pg)


def _apply(wv, x_parts, ea_parts, u_parts, idx2, cidx2s, N, E, prep_base=None):
    if prep_base is None:
        prep = _mm(x_parts, wv["W_prep"])
    else:
        prep = _mm([x_parts[-1]], [wv["W_prep"][-1]], base=prep_base)
    xs = prep[:, :LAT]
    xd = prep[:, LAT:2 * LAT]
    hx = prep[:, 2 * LAT:]
    cvec_e, cvec_n = _cvecs_call(u_parts, wv["W_eu"], wv["V_u"],
                                 wv["pe"]["b1"].reshape(1, LAT),
                                 wv["pn"]["b1"].reshape(1, LAT))
    gs, gd = _sc_gather2(xs, xd, idx2, E)
    e = _edge_mlp(gs, gd, ea_parts, wv["W_ea"], cvec_e,
                  wv["pe"]["w2"], wv["pe"]["b2"].reshape(1, -1))
    aggp = _sc_scatter_add(e, cidx2s, N)
    xn = _node_mlp(hx, aggp, wv["V_a"], cvec_n,
                   wv["pn"]["w2"], wv["pn"]["b2"].reshape(1, -1))
    gu = _global_mlp(xn, u_parts, wv["G_m"], wv["G_u"],
                     wv["pg"]["b1"].reshape(1, LAT),
                     wv["pg"]["w2"], wv["pg"]["b2"].reshape(1, -1))
    return xn, e, gu


def _cvecs_call(u_parts, we_parts, wn_parts, b1e, b1n):
    nu = len(u_parts)
    return pl.pallas_call(
        functools.partial(_cvec_body, nu),
        in_specs=(
            [pl.BlockSpec((1, u.shape[1]), lambda: (0, 0)) for u in u_parts]
            + [pl.BlockSpec((w.shape[0], LAT), lambda: (0, 0)) for w in we_parts]
            + [pl.BlockSpec((w.shape[0], LAT), lambda: (0, 0)) for w in wn_parts]
            + [pl.BlockSpec((1, LAT), lambda: (0, 0)),
               pl.BlockSpec((1, LAT), lambda: (0, 0))]
        ),
        out_specs=[pl.BlockSpec((1, LAT), lambda: (0, 0)),
                   pl.BlockSpec((1, LAT), lambda: (0, 0))],
        out_shape=[jax.ShapeDtypeStruct((1, LAT), _F32),
                   jax.ShapeDtypeStruct((1, LAT), _F32)],
    )(*u_parts, *we_parts, *wn_parts, b1e, b1n)


def kernel(x, edge_attr, global_attr, params, edge_index):
    row, col = edge_index[0], edge_index[1]
    N = x.shape[0]
    E = edge_attr.shape[0]
    DN, DE, DG = x.shape[1], edge_attr.shape[1], global_attr.shape[1]

    wv_enc = _weight_views(params["encoder"], [DN], [DE], [DG], DE)
    wv_core = _weight_views(params["core"], [DN, DN], [DE, DE], [DG, DG], DE)
    wv_dec = _weight_views(params["decoder"], [DN], [DE], [DG], DE)

    idx2 = jnp.concatenate([row.reshape(-1, _GWG), col.reshape(-1, _GWG)],
                           axis=1)
    cidx2s = col.reshape(-1, _GWS)

    x1, e1, u1 = _apply(wv_enc, [x], [edge_attr], [global_attr],
                        idx2, cidx2s, N, E)
    x0, e0, u0 = x1, e1, u1

    prep_base0 = _mm([x0], [wv_core["W_prep"][0]])
    xc, ec, uc = x1, e1, u1
    for _ in range(5):
        xc, ec, uc = _apply(wv_core, [x0, xc], [e0, ec], [u0, uc],
                            idx2, cidx2s, N, E, prep_base=prep_base0)

    return _apply(wv_dec, [xc], [ec], [uc], idx2, cidx2s, N, E)


# speedup vs baseline: 4.2922x; 1.0164x over previous
import functools

import jax
import jax.numpy as jnp
import numpy as np
from jax import lax
from jax.experimental import pallas as pl
from jax.experimental.pallas import tpu as pltpu
from jax.experimental.pallas import tpu_sc as plsc

_NC = 2
_NS = 16
_NW = _NC * _NS

LAT = 128
_EB = 2000
_NB = 2000
_F32 = jnp.float32


def _split_rows(W, dims):
    out, o = [], 0
    for d in dims:
        out.append(W[o:o + d])
        o += d
    return out


def _mm3_body(has_base, na, *refs):
    a = refs[:na]
    w = refs[na:2 * na]
    acc = jnp.dot(a[0][...], w[0][...], preferred_element_type=_F32)
    for i in range(1, na):
        acc = acc + jnp.dot(a[i][...], w[i][...], preferred_element_type=_F32)
    k = 2 * na
    for j in range(3):
        part = acc[:, j * LAT:(j + 1) * LAT]
        if has_base:
            part = part + refs[k + j][...]
        refs[-3 + j][...] = part


def _mm3(as_, ws, base3=None, block=_NB):
    R = as_[0].shape[0]
    K = ws[0].shape[1]
    na = len(as_)
    in_specs = (
        [pl.BlockSpec((block, a.shape[1]), lambda i: (i, 0)) for a in as_]
        + [pl.BlockSpec((w.shape[0], K), lambda i: (0, 0)) for w in ws]
    )
    args = list(as_) + list(ws)
    if base3 is not None:
        in_specs += [pl.BlockSpec((block, LAT), lambda i: (i, 0))] * 3
        args += list(base3)
    out = pl.pallas_call(
        functools.partial(_mm3_body, base3 is not None, na),
        grid=(R // block,),
        in_specs=in_specs,
        out_specs=[pl.BlockSpec((block, LAT), lambda i: (i, 0))] * 3,
        out_shape=[jax.ShapeDtypeStruct((R, LAT), _F32)] * 3,
    )(*args)
    return tuple(out)


def _edge_body(ne, *refs):
    acc = refs[0][...] + refs[1][...] + refs[2 + 2 * ne][...]
    for i in range(ne):
        acc = acc + jnp.dot(refs[2 + i][...], refs[2 + ne + i][...],
                            preferred_element_type=_F32)
    h = jnp.maximum(acc, 0.0)
    refs[-1][...] = (jnp.dot(h, refs[3 + 2 * ne][...],
                             preferred_element_type=_F32) + refs[4 + 2 * ne][...])


def _edge_mlp(gs, gd, ea_parts, we_parts, cvec, w2, b2):
    E = gs.shape[0]
    ne = len(ea_parts)
    d_out = w2.shape[1]
    in_specs = (
        [pl.BlockSpec((_EB, LAT), lambda i: (i, 0)),
         pl.BlockSpec((_EB, LAT), lambda i: (i, 0))]
        + [pl.BlockSpec((_EB, ea.shape[1]), lambda i: (i, 0)) for ea in ea_parts]
        + [pl.BlockSpec((we.shape[0], LAT), lambda i: (0, 0)) for we in we_parts]
        + [pl.BlockSpec((1, LAT), lambda i: (0, 0)),
           pl.BlockSpec((LAT, d_out), lambda i: (0, 0)),
           pl.BlockSpec((1, d_out), lambda i: (0, 0))]
    )
    return pl.pallas_call(
        functools.partial(_edge_body, ne),
        grid=(E // _EB,),
        in_specs=in_specs,
        out_specs=pl.BlockSpec((_EB, d_out), lambda i: (i, 0)),
        out_shape=jax.ShapeDtypeStruct((E, d_out), _F32),
    )(gs, gd, *ea_parts, *we_parts, cvec, w2, b2)


def _node_body(hx, aggp, va, cvec, v2, b2, out):
    agg = aggp[0] + aggp[1]
    h = jnp.maximum(hx[...] + jnp.dot(agg, va[...],
                                      preferred_element_type=_F32) + cvec[...], 0.0)
    out[...] = jnp.dot(h, v2[...], preferred_element_type=_F32) + b2[...]


def _node_mlp(hx, aggp, va, cvec, v2, b2):
    N = hx.shape[0]
    da = aggp.shape[2]
    d_out = v2.shape[1]
    return pl.pallas_call(
        _node_body,
        grid=(N // _NB,),
        in_specs=[
            pl.BlockSpec((_NB, LAT), lambda i: (i, 0)),
            pl.BlockSpec((_NC, _NB, da), lambda i: (0, i, 0)),
            pl.BlockSpec((da, LAT), lambda i: (0, 0)),
            pl.BlockSpec((1, LAT), lambda i: (0, 0)),
            pl.BlockSpec((LAT, d_out), lambda i: (0, 0)),
            pl.BlockSpec((1, d_out), lambda i: (0, 0)),
        ],
        out_specs=pl.BlockSpec((_NB, d_out), lambda i: (i, 0)),
        out_shape=jax.ShapeDtypeStruct((N, d_out), _F32),
    )(hx, aggp, va, cvec, v2, b2)


def _global_body(nu, inv_n, *refs):
    m = jnp.sum(refs[0][...], axis=0, keepdims=True) * inv_n
    acc = jnp.dot(m, refs[1 + nu][...], preferred_element_type=_F32)
    for i in range(nu):
        acc = acc + jnp.dot(refs[1 + i][...], refs[2 + nu + i][...],
                            preferred_element_type=_F32)
    h = jnp.maximum(acc + refs[2 + 2 * nu][...], 0.0)
    refs[-1][...] = (jnp.dot(h, refs[3 + 2 * nu][...],
                             preferred_element_type=_F32) + refs[4 + 2 * nu][...])


def _global_mlp(xn, u_parts, gm, gu_parts, b1, g2, b2):
    N = xn.shape[0]
    nu = len(u_parts)
    d_out = g2.shape[1]
    in_specs = (
        [pl.BlockSpec((N, LAT), lambda: (0, 0))]
        + [pl.BlockSpec((1, u.shape[1]), lambda: (0, 0)) for u in u_parts]
        + [pl.BlockSpec((LAT, LAT), lambda: (0, 0))]
        + [pl.BlockSpec((w.shape[0], LAT), lambda: (0, 0)) for w in gu_parts]
        + [pl.BlockSpec((1, LAT), lambda: (0, 0)),
           pl.BlockSpec((LAT, d_out), lambda: (0, 0)),
           pl.BlockSpec((1, d_out), lambda: (0, 0))]
    )
    return pl.pallas_call(
        functools.partial(_global_body, nu, 1.0 / N),
        in_specs=in_specs,
        out_specs=pl.BlockSpec((1, d_out), lambda: (0, 0)),
        out_shape=jax.ShapeDtypeStruct((1, d_out), _F32),
    )(xn, *u_parts, gm, *gu_parts, b1, g2, b2)


def _cvec_body(nu, *refs):
    acc_e = refs[3 * nu][...]
    acc_n = refs[3 * nu + 1][...]
    for i in range(nu):
        u = refs[i][...]
        acc_e = acc_e + jnp.dot(u, refs[nu + i][...], preferred_element_type=_F32)
        acc_n = acc_n + jnp.dot(u, refs[2 * nu + i][...],
                                preferred_element_type=_F32)
    refs[-2][...] = acc_e
    refs[-1][...] = acc_n


_GWG = 200
_GWS = 2000


def _gather_body(nwin, xs_hbm, xd_hbm, idx_hbm, gs_hbm, gd_hbm,
                 ix0, ix1, a0, b0, a1, b1, si0, si1, sg0, sg1, sw0, sw1):
    wid = lax.axis_index("s") * _NC + lax.axis_index("c")
    base = wid * nwin
    ix = (ix0, ix1)
    ab = ((a0, b0), (a1, b1))
    si = (si0, si1)
    sg = (sg0, sg1)
    sw = (sw0, sw1)

    def prefetch(w, slot):
        pltpu.async_copy(idx_hbm.at[base + w], ix[slot], si[slot])

    def window(w, slot, first):
        a, b = ab[slot]
        if not first:
            pltpu.make_async_copy(a, gs_hbm.at[pl.ds(0, _GWG)], sw[slot]).wait()
            pltpu.make_async_copy(b, gd_hbm.at[pl.ds(0, _GWG)], sw[slot]).wait()
        pltpu.make_async_copy(idx_hbm.at[base], ix[slot], si[slot]).wait()
        cpa = pltpu.async_copy(xs_hbm.at[ix[slot].at[pl.ds(0, _GWG)]], a, sg[slot])
        cpb = pltpu.async_copy(xd_hbm.at[ix[slot].at[pl.ds(_GWG, _GWG)]], b, sg[slot])
        cpa.wait()
        cpb.wait()
        if isinstance(w, int):
            if w + 2 < nwin:
                prefetch(w + 2, slot)
        else:
            @pl.when(w + 2 < nwin)
            def _():
                prefetch(w + 2, slot)
        pltpu.async_copy(a, gs_hbm.at[pl.ds((base + w) * _GWG, _GWG)], sw[slot])
        pltpu.async_copy(b, gd_hbm.at[pl.ds((base + w) * _GWG, _GWG)], sw[slot])

    prefetch(0, 0)
    prefetch(1, 1)
    window(0, 0, True)
    window(1, 1, True)

    def step(k, carry):
        window(2 * k, 0, False)
        window(2 * k + 1, 1, False)
        return carry

    lax.fori_loop(1, nwin // 2, step, 0)
    for slot in (0, 1):
        pltpu.make_async_copy(ab[slot][0], gs_hbm.at[pl.ds(0, _GWG)], sw[slot]).wait()
        pltpu.make_async_copy(ab[slot][1], gd_hbm.at[pl.ds(0, _GWG)], sw[slot]).wait()


def _sc_gather2(xs, xd, idx2, E):
    nwin = idx2.shape[0] // _NW
    mesh = plsc.VectorSubcoreMesh(core_axis_name="c", subcore_axis_name="s",
                                  num_cores=_NC, num_subcores=_NS)
    fn = pl.kernel(
        functools.partial(_gather_body, nwin),
        out_type=[jax.ShapeDtypeStruct((E, LAT), _F32),
                  jax.ShapeDtypeStruct((E, LAT), _F32)],
        mesh=mesh,
        scratch_types=[
            pltpu.VMEM((2 * _GWG,), jnp.int32),
            pltpu.VMEM((2 * _GWG,), jnp.int32),
            pltpu.VMEM((_GWG, LAT), _F32),
            pltpu.VMEM((_GWG, LAT), _F32),
            pltpu.VMEM((_GWG, LAT), _F32),
            pltpu.VMEM((_GWG, LAT), _F32),
            pltpu.SemaphoreType.DMA,
            pltpu.SemaphoreType.DMA,
            pltpu.SemaphoreType.DMA,
            pltpu.SemaphoreType.DMA,
            pltpu.SemaphoreType.DMA,
            pltpu.SemaphoreType.DMA,
        ],
    )
    return fn(xs, xd, idx2)


def _scatter_body(nwin, nchunk, e_hbm, cidx_hbm, zero_hbm, out_hbm,
                  idx_v, upd_v, obuf, acc):
    cid = lax.axis_index("c")
    sid = lax.axis_index("s")
    wid = sid * _NC + cid
    nrows = acc.shape[0]

    @pl.when(sid * nchunk < nrows)
    def _():
        pltpu.sync_copy(zero_hbm.at[pl.ds(sid * nchunk, nchunk)], obuf)
        pltpu.sync_copy(obuf, acc.at[pl.ds(sid * nchunk, nchunk)])

    plsc.subcore_barrier()

    def step(j, carry):
        r = wid * nwin + j
        pltpu.sync_copy(cidx_hbm.at[r], idx_v)
        pltpu.sync_copy(e_hbm.at[pl.ds(r * _GWS, _GWS)], upd_v)
        pltpu.sync_copy(upd_v, acc.at[idx_v], add=True)
        return carry

    lax.fori_loop(0, nwin, step, 0)
    plsc.subcore_barrier()

    @pl.when(sid * nchunk < nrows)
    def _():
        pltpu.sync_copy(acc.at[pl.ds(sid * nchunk, nchunk)], obuf)
        pltpu.sync_copy(obuf, out_hbm.at[cid, pl.ds(sid * nchunk, nchunk)])


def _sc_scatter_add(e, cidx2, N):
    E, de = e.shape
    nwin = cidx2.shape[0] // _NW
    nchunk = 1000
    mesh = plsc.VectorSubcoreMesh(core_axis_name="c", subcore_axis_name="s",
                                  num_cores=_NC, num_subcores=_NS)
    fn = pl.kernel(
        functools.partial(_scatter_body, nwin, nchunk),
        out_type=jax.ShapeDtypeStruct((_NC, N, de), _F32),
        mesh=mesh,
        compiler_params=pltpu.CompilerParams(use_tc_tiling_on_sc=False),
        scratch_types=[
            pltpu.VMEM((_GWS,), jnp.int32),
            pltpu.VMEM((_GWS, de), _F32),
            pltpu.VMEM((1000, de), _F32),
            pltpu.VMEM_SHARED((N, de), _F32),
        ],
    )
    return fn(e, cidx2, jnp.zeros((N, de), _F32))


def _weight_views(p, dims_x, dims_e, dims_u, de_out):
    pe, pn, pg = p["edge"], p["node"], p["global"]
    nx, nee, nuu = len(dims_x), len(dims_e), len(dims_u)
    parts = _split_rows(pe["w1"], dims_x + dims_x + dims_e + dims_u)
    W_src = parts[:nx]
    W_dst = parts[nx:2 * nx]
    W_ea = parts[2 * nx:2 * nx + nee]
    W_eu = parts[2 * nx + nee:]
    parts = _split_rows(pn["w1"], dims_x + [de_out] + dims_u)
    V_x = parts[:nx]
    V_a = parts[nx]
    V_u = parts[nx + 1:]
    parts = _split_rows(pg["w1"], [LAT] + dims_u)
    G_m = parts[0]
    G_u = parts[1:]
    W_prep = [jnp.concatenate([W_src[i], W_dst[i], V_x[i]], axis=1)
              for i in range(nx)]
    return dict(W_prep=W_prep, W_ea=W_ea, W_eu=W_eu, V_a=V_a, V_u=V_u,
                G_m=G_m, G_u=G_u, pe=pe, pn=pn, pg=pg)


def _apply(wv, x_parts, ea_parts, u_parts, idx2, cidx2s, N, E, prep_base=None):
    if prep_base is None:
        xs, xd, hx = _mm3(x_parts, wv["W_prep"])
    else:
        xs, xd, hx = _mm3([x_parts[-1]], [wv["W_prep"][-1]], base3=prep_base)
    cvec_e, cvec_n = _cvecs_call(u_parts, wv["W_eu"], wv["V_u"],
                                 wv["pe"]["b1"].reshape(1, LAT),
                                 wv["pn"]["b1"].reshape(1, LAT))
    gs, gd = _sc_gather2(xs, xd, idx2, E)
    e = _edge_mlp(gs, gd, ea_parts, wv["W_ea"], cvec_e,
                  wv["pe"]["w2"], wv["pe"]["b2"].reshape(1, -1))
    aggp = _sc_scatter_add(e, cidx2s, N)
    xn = _node_mlp(hx, aggp, wv["V_a"], cvec_n,
                   wv["pn"]["w2"], wv["pn"]["b2"].reshape(1, -1))
    gu = _global_mlp(xn, u_parts, wv["G_m"], wv["G_u"],
                     wv["pg"]["b1"].reshape(1, LAT),
                     wv["pg"]["w2"], wv["pg"]["b2"].reshape(1, -1))
    return xn, e, gu


def _cvecs_call(u_parts, we_parts, wn_parts, b1e, b1n):
    nu = len(u_parts)
    return pl.pallas_call(
        functools.partial(_cvec_body, nu),
        in_specs=(
            [pl.BlockSpec((1, u.shape[1]), lambda: (0, 0)) for u in u_parts]
            + [pl.BlockSpec((w.shape[0], LAT), lambda: (0, 0)) for w in we_parts]
            + [pl.BlockSpec((w.shape[0], LAT), lambda: (0, 0)) for w in wn_parts]
            + [pl.BlockSpec((1, LAT), lambda: (0, 0)),
               pl.BlockSpec((1, LAT), lambda: (0, 0))]
        ),
        out_specs=[pl.BlockSpec((1, LAT), lambda: (0, 0)),
                   pl.BlockSpec((1, LAT), lambda: (0, 0))],
        out_shape=[jax.ShapeDtypeStruct((1, LAT), _F32),
                   jax.ShapeDtypeStruct((1, LAT), _F32)],
    )(*u_parts, *we_parts, *wn_parts, b1e, b1n)


def kernel(x, edge_attr, global_attr, params, edge_index):
    row, col = edge_index[0], edge_index[1]
    N = x.shape[0]
    E = edge_attr.shape[0]
    DN, DE, DG = x.shape[1], edge_attr.shape[1], global_attr.shape[1]

    wv_enc = _weight_views(params["encoder"], [DN], [DE], [DG], DE)
    wv_core = _weight_views(params["core"], [DN, DN], [DE, DE], [DG, DG], DE)
    wv_dec = _weight_views(params["decoder"], [DN], [DE], [DG], DE)

    idx2 = jnp.concatenate([row.reshape(-1, _GWG), col.reshape(-1, _GWG)],
                           axis=1)
    cidx2s = col.reshape(-1, _GWS)

    x1, e1, u1 = _apply(wv_enc, [x], [edge_attr], [global_attr],
                        idx2, cidx2s, N, E)
    x0, e0, u0 = x1, e1, u1

    prep_base0 = _mm3([x0], [wv_core["W_prep"][0]])
    xc, ec, uc = x1, e1, u1
    for _ in range(5):
        xc, ec, uc = _apply(wv_core, [x0, xc], [e0, ec], [u0, uc],
                            idx2, cidx2s, N, E, prep_base=prep_base0)

    return _apply(wv_dec, [xc], [ec], [uc], idx2, cidx2s, N, E)


# speedup vs baseline: 4.5020x; 1.0489x over previous
import functools

import jax
import jax.numpy as jnp
import numpy as np
from jax import lax
from jax.experimental import pallas as pl
from jax.experimental.pallas import tpu as pltpu
from jax.experimental.pallas import tpu_sc as plsc

_NC = 2
_NS = 16
_NW = _NC * _NS

LAT = 128
_EB = 4000
_NB = 2000
_F32 = jnp.float32


def _split_rows(W, dims):
    out, o = [], 0
    for d in dims:
        out.append(W[o:o + d])
        o += d
    return out


def _mm3_body(has_base, na, *refs):
    a = refs[:na]
    w = refs[na:2 * na]
    acc = jnp.dot(a[0][...], w[0][...], preferred_element_type=_F32)
    for i in range(1, na):
        acc = acc + jnp.dot(a[i][...], w[i][...], preferred_element_type=_F32)
    k = 2 * na
    for j in range(3):
        part = acc[:, j * LAT:(j + 1) * LAT]
        if has_base:
            part = part + refs[k + j][...]
        refs[-3 + j][...] = part


def _mm3(as_, ws, base3=None, block=_NB):
    R = as_[0].shape[0]
    K = ws[0].shape[1]
    na = len(as_)
    in_specs = (
        [pl.BlockSpec((block, a.shape[1]), lambda i: (i, 0)) for a in as_]
        + [pl.BlockSpec((w.shape[0], K), lambda i: (0, 0)) for w in ws]
    )
    args = list(as_) + list(ws)
    if base3 is not None:
        in_specs += [pl.BlockSpec((block, LAT), lambda i: (i, 0))] * 3
        args += list(base3)
    out = pl.pallas_call(
        functools.partial(_mm3_body, base3 is not None, na),
        grid=(R // block,),
        in_specs=in_specs,
        out_specs=[pl.BlockSpec((block, LAT), lambda i: (i, 0))] * 3,
        out_shape=[jax.ShapeDtypeStruct((R, LAT), _F32)] * 3,
    )(*args)
    return tuple(out)


def _edge_body(ne, *refs):
    acc = refs[0][...] + refs[1][...] + refs[2 + 2 * ne][...]
    for i in range(ne):
        acc = acc + jnp.dot(refs[2 + i][...], refs[2 + ne + i][...],
                            preferred_element_type=_F32)
    h = jnp.maximum(acc, 0.0)
    refs[-1][...] = (jnp.dot(h, refs[3 + 2 * ne][...],
                             preferred_element_type=_F32) + refs[4 + 2 * ne][...])


def _edge_mlp(gs, gd, ea_parts, we_parts, cvec, w2, b2):
    E = gs.shape[0]
    ne = len(ea_parts)
    d_out = w2.shape[1]
    in_specs = (
        [pl.BlockSpec((_EB, LAT), lambda i: (i, 0)),
         pl.BlockSpec((_EB, LAT), lambda i: (i, 0))]
        + [pl.BlockSpec((_EB, ea.shape[1]), lambda i: (i, 0)) for ea in ea_parts]
        + [pl.BlockSpec((we.shape[0], LAT), lambda i: (0, 0)) for we in we_parts]
        + [pl.BlockSpec((1, LAT), lambda i: (0, 0)),
           pl.BlockSpec((LAT, d_out), lambda i: (0, 0)),
           pl.BlockSpec((1, d_out), lambda i: (0, 0))]
    )
    return pl.pallas_call(
        functools.partial(_edge_body, ne),
        grid=(E // _EB,),
        in_specs=in_specs,
        out_specs=pl.BlockSpec((_EB, d_out), lambda i: (i, 0)),
        out_shape=jax.ShapeDtypeStruct((E, d_out), _F32),
    )(gs, gd, *ea_parts, *we_parts, cvec, w2, b2)


def _node_body(hx, aggp, va, cvec, v2, b2, out):
    agg = aggp[0] + aggp[1]
    h = jnp.maximum(hx[...] + jnp.dot(agg, va[...],
                                      preferred_element_type=_F32) + cvec[...], 0.0)
    out[...] = jnp.dot(h, v2[...], preferred_element_type=_F32) + b2[...]


def _node_mlp(hx, aggp, va, cvec, v2, b2):
    N = hx.shape[0]
    da = aggp.shape[2]
    d_out = v2.shape[1]
    return pl.pallas_call(
        _node_body,
        grid=(N // _NB,),
        in_specs=[
            pl.BlockSpec((_NB, LAT), lambda i: (i, 0)),
            pl.BlockSpec((_NC, _NB, da), lambda i: (0, i, 0)),
            pl.BlockSpec((da, LAT), lambda i: (0, 0)),
            pl.BlockSpec((1, LAT), lambda i: (0, 0)),
            pl.BlockSpec((LAT, d_out), lambda i: (0, 0)),
            pl.BlockSpec((1, d_out), lambda i: (0, 0)),
        ],
        out_specs=pl.BlockSpec((_NB, d_out), lambda i: (i, 0)),
        out_shape=jax.ShapeDtypeStruct((N, d_out), _F32),
    )(hx, aggp, va, cvec, v2, b2)


def _global_body(nu, inv_n, *refs):
    m = jnp.sum(refs[0][...], axis=0, keepdims=True) * inv_n
    acc = jnp.dot(m, refs[1 + nu][...], preferred_element_type=_F32)
    for i in range(nu):
        acc = acc + jnp.dot(refs[1 + i][...], refs[2 + nu + i][...],
                            preferred_element_type=_F32)
    h = jnp.maximum(acc + refs[2 + 2 * nu][...], 0.0)
    refs[-1][...] = (jnp.dot(h, refs[3 + 2 * nu][...],
                             preferred_element_type=_F32) + refs[4 + 2 * nu][...])


def _global_mlp(xn, u_parts, gm, gu_parts, b1, g2, b2):
    N = xn.shape[0]
    nu = len(u_parts)
    d_out = g2.shape[1]
    in_specs = (
        [pl.BlockSpec((N, LAT), lambda: (0, 0))]
        + [pl.BlockSpec((1, u.shape[1]), lambda: (0, 0)) for u in u_parts]
        + [pl.BlockSpec((LAT, LAT), lambda: (0, 0))]
        + [pl.BlockSpec((w.shape[0], LAT), lambda: (0, 0)) for w in gu_parts]
        + [pl.BlockSpec((1, LAT), lambda: (0, 0)),
           pl.BlockSpec((LAT, d_out), lambda: (0, 0)),
           pl.BlockSpec((1, d_out), lambda: (0, 0))]
    )
    return pl.pallas_call(
        functools.partial(_global_body, nu, 1.0 / N),
        in_specs=in_specs,
        out_specs=pl.BlockSpec((1, d_out), lambda: (0, 0)),
        out_shape=jax.ShapeDtypeStruct((1, d_out), _F32),
    )(xn, *u_parts, gm, *gu_parts, b1, g2, b2)


def _cvec_body(nu, *refs):
    acc_e = refs[3 * nu][...]
    acc_n = refs[3 * nu + 1][...]
    for i in range(nu):
        u = refs[i][...]
        acc_e = acc_e + jnp.dot(u, refs[nu + i][...], preferred_element_type=_F32)
        acc_n = acc_n + jnp.dot(u, refs[2 * nu + i][...],
                                preferred_element_type=_F32)
    refs[-2][...] = acc_e
    refs[-1][...] = acc_n


_GWG = 200
_GWS = 2000


def _gather_body(nwin, xs_hbm, xd_hbm, idx_hbm, gs_hbm, gd_hbm,
                 ix0, ix1, a0, b0, a1, b1, si0, si1, sg0, sg1, sw0, sw1):
    wid = lax.axis_index("s") * _NC + lax.axis_index("c")
    base = wid * nwin
    ix = (ix0, ix1)
    ab = ((a0, b0), (a1, b1))
    si = (si0, si1)
    sg = (sg0, sg1)
    sw = (sw0, sw1)

    def prefetch(w, slot):
        pltpu.async_copy(idx_hbm.at[base + w], ix[slot], si[slot])

    def window(w, slot, first):
        a, b = ab[slot]
        if not first:
            pltpu.make_async_copy(a, gs_hbm.at[pl.ds(0, _GWG)], sw[slot]).wait()
            pltpu.make_async_copy(b, gd_hbm.at[pl.ds(0, _GWG)], sw[slot]).wait()
        pltpu.make_async_copy(idx_hbm.at[base], ix[slot], si[slot]).wait()
        cpa = pltpu.async_copy(xs_hbm.at[ix[slot].at[pl.ds(0, _GWG)]], a, sg[slot])
        cpb = pltpu.async_copy(xd_hbm.at[ix[slot].at[pl.ds(_GWG, _GWG)]], b, sg[slot])
        cpa.wait()
        cpb.wait()
        if isinstance(w, int):
            if w + 2 < nwin:
                prefetch(w + 2, slot)
        else:
            @pl.when(w + 2 < nwin)
            def _():
                prefetch(w + 2, slot)
        pltpu.async_copy(a, gs_hbm.at[pl.ds((base + w) * _GWG, _GWG)], sw[slot])
        pltpu.async_copy(b, gd_hbm.at[pl.ds((base + w) * _GWG, _GWG)], sw[slot])

    prefetch(0, 0)
    prefetch(1, 1)
    window(0, 0, True)
    window(1, 1, True)

    def step(k, carry):
        window(2 * k, 0, False)
        window(2 * k + 1, 1, False)
        return carry

    lax.fori_loop(1, nwin // 2, step, 0)
    for slot in (0, 1):
        pltpu.make_async_copy(ab[slot][0], gs_hbm.at[pl.ds(0, _GWG)], sw[slot]).wait()
        pltpu.make_async_copy(ab[slot][1], gd_hbm.at[pl.ds(0, _GWG)], sw[slot]).wait()


def _sc_gather2(xs, xd, idx2, E):
    nwin = idx2.shape[0] // _NW
    mesh = plsc.VectorSubcoreMesh(core_axis_name="c", subcore_axis_name="s",
                                  num_cores=_NC, num_subcores=_NS)
    fn = pl.kernel(
        functools.partial(_gather_body, nwin),
        out_type=[jax.ShapeDtypeStruct((E, LAT), _F32),
                  jax.ShapeDtypeStruct((E, LAT), _F32)],
        mesh=mesh,
        scratch_types=[
            pltpu.VMEM((2 * _GWG,), jnp.int32),
            pltpu.VMEM((2 * _GWG,), jnp.int32),
            pltpu.VMEM((_GWG, LAT), _F32),
            pltpu.VMEM((_GWG, LAT), _F32),
            pltpu.VMEM((_GWG, LAT), _F32),
            pltpu.VMEM((_GWG, LAT), _F32),
            pltpu.SemaphoreType.DMA,
            pltpu.SemaphoreType.DMA,
            pltpu.SemaphoreType.DMA,
            pltpu.SemaphoreType.DMA,
            pltpu.SemaphoreType.DMA,
            pltpu.SemaphoreType.DMA,
        ],
    )
    return fn(xs, xd, idx2)


def _scatter_body(nwin, nchunk, e_hbm, cidx_hbm, zero_hbm, out_hbm,
                  idx_v, upd_v, obuf, acc):
    cid = lax.axis_index("c")
    sid = lax.axis_index("s")
    wid = sid * _NC + cid
    nrows = acc.shape[0]

    @pl.when(sid * nchunk < nrows)
    def _():
        pltpu.sync_copy(zero_hbm.at[pl.ds(sid * nchunk, nchunk)], obuf)
        pltpu.sync_copy(obuf, acc.at[pl.ds(sid * nchunk, nchunk)])

    plsc.subcore_barrier()

    def step(j, carry):
        r = wid * nwin + j
        pltpu.sync_copy(cidx_hbm.at[r], idx_v)
        pltpu.sync_copy(e_hbm.at[pl.ds(r * _GWS, _GWS)], upd_v)
        pltpu.sync_copy(upd_v, acc.at[idx_v], add=True)
        return carry

    lax.fori_loop(0, nwin, step, 0)
    plsc.subcore_barrier()

    @pl.when(sid * nchunk < nrows)
    def _():
        pltpu.sync_copy(acc.at[pl.ds(sid * nchunk, nchunk)], obuf)
        pltpu.sync_copy(obuf, out_hbm.at[cid, pl.ds(sid * nchunk, nchunk)])


def _sc_scatter_add(e, cidx2, N):
    E, de = e.shape
    nwin = cidx2.shape[0] // _NW
    nchunk = 1000
    mesh = plsc.VectorSubcoreMesh(core_axis_name="c", subcore_axis_name="s",
                                  num_cores=_NC, num_subcores=_NS)
    fn = pl.kernel(
        functools.partial(_scatter_body, nwin, nchunk),
        out_type=jax.ShapeDtypeStruct((_NC, N, de), _F32),
        mesh=mesh,
        compiler_params=pltpu.CompilerParams(use_tc_tiling_on_sc=False),
        scratch_types=[
            pltpu.VMEM((_GWS,), jnp.int32),
            pltpu.VMEM((_GWS, de), _F32),
            pltpu.VMEM((1000, de), _F32),
            pltpu.VMEM_SHARED((N, de), _F32),
        ],
    )
    return fn(e, cidx2, jnp.zeros((N, de), _F32))


def _weight_views(p, dims_x, dims_e, dims_u, de_out):
    pe, pn, pg = p["edge"], p["node"], p["global"]
    nx, nee, nuu = len(dims_x), len(dims_e), len(dims_u)
    parts = _split_rows(pe["w1"], dims_x + dims_x + dims_e + dims_u)
    W_src = parts[:nx]
    W_dst = parts[nx:2 * nx]
    W_ea = parts[2 * nx:2 * nx + nee]
    W_eu = parts[2 * nx + nee:]
    parts = _split_rows(pn["w1"], dims_x + [de_out] + dims_u)
    V_x = parts[:nx]
    V_a = parts[nx]
    V_u = parts[nx + 1:]
    parts = _split_rows(pg["w1"], [LAT] + dims_u)
    G_m = parts[0]
    G_u = parts[1:]
    W_prep = [jnp.concatenate([W_src[i], W_dst[i], V_x[i]], axis=1)
              for i in range(nx)]
    return dict(W_prep=W_prep, W_ea=W_ea, W_eu=W_eu, V_a=V_a, V_u=V_u,
                G_m=G_m, G_u=G_u, pe=pe, pn=pn, pg=pg)


def _apply(wv, x_parts, ea_parts, u_parts, idx2, cidx2s, N, E, prep_base=None):
    if prep_base is None:
        xs, xd, hx = _mm3(x_parts, wv["W_prep"])
    else:
        xs, xd, hx = _mm3([x_parts[-1]], [wv["W_prep"][-1]], base3=prep_base)
    cvec_e, cvec_n = _cvecs_call(u_parts, wv["W_eu"], wv["V_u"],
                                 wv["pe"]["b1"].reshape(1, LAT),
                                 wv["pn"]["b1"].reshape(1, LAT))
    gs, gd = _sc_gather2(xs, xd, idx2, E)
    e = _edge_mlp(gs, gd, ea_parts, wv["W_ea"], cvec_e,
                  wv["pe"]["w2"], wv["pe"]["b2"].reshape(1, -1))
    aggp = _sc_scatter_add(e, cidx2s, N)
    xn = _node_mlp(hx, aggp, wv["V_a"], cvec_n,
                   wv["pn"]["w2"], wv["pn"]["b2"].reshape(1, -1))
    gu = _global_mlp(xn, u_parts, wv["G_m"], wv["G_u"],
                     wv["pg"]["b1"].reshape(1, LAT),
                     wv["pg"]["w2"], wv["pg"]["b2"].reshape(1, -1))
    return xn, e, gu


def _cvecs_call(u_parts, we_parts, wn_parts, b1e, b1n):
    nu = len(u_parts)
    return pl.pallas_call(
        functools.partial(_cvec_body, nu),
        in_specs=(
            [pl.BlockSpec((1, u.shape[1]), lambda: (0, 0)) for u in u_parts]
            + [pl.BlockSpec((w.shape[0], LAT), lambda: (0, 0)) for w in we_parts]
            + [pl.BlockSpec((w.shape[0], LAT), lambda: (0, 0)) for w in wn_parts]
            + [pl.BlockSpec((1, LAT), lambda: (0, 0)),
               pl.BlockSpec((1, LAT), lambda: (0, 0))]
        ),
        out_specs=[pl.BlockSpec((1, LAT), lambda: (0, 0)),
                   pl.BlockSpec((1, LAT), lambda: (0, 0))],
        out_shape=[jax.ShapeDtypeStruct((1, LAT), _F32),
                   jax.ShapeDtypeStruct((1, LAT), _F32)],
    )(*u_parts, *we_parts, *wn_parts, b1e, b1n)


def kernel(x, edge_attr, global_attr, params, edge_index):
    row, col = edge_index[0], edge_index[1]
    N = x.shape[0]
    E = edge_attr.shape[0]
    DN, DE, DG = x.shape[1], edge_attr.shape[1], global_attr.shape[1]

    wv_enc = _weight_views(params["encoder"], [DN], [DE], [DG], DE)
    wv_core = _weight_views(params["core"], [DN, DN], [DE, DE], [DG, DG], DE)
    wv_dec = _weight_views(params["decoder"], [DN], [DE], [DG], DE)

    idx2 = jnp.concatenate([row.reshape(-1, _GWG), col.reshape(-1, _GWG)],
                           axis=1)
    cidx2s = col.reshape(-1, _GWS)

    x1, e1, u1 = _apply(wv_enc, [x], [edge_attr], [global_attr],
                        idx2, cidx2s, N, E)
    x0, e0, u0 = x1, e1, u1

    prep_base0 = _mm3([x0], [wv_core["W_prep"][0]])
    xc, ec, uc = x1, e1, u1
    for _ in range(5):
        xc, ec, uc = _apply(wv_core, [x0, xc], [e0, ec], [u0, uc],
                            idx2, cidx2s, N, E, prep_base=prep_base0)

    return _apply(wv_dec, [xc], [ec], [uc], idx2, cidx2s, N, E)


# speedup vs baseline: 4.5194x; 1.0039x over previous
import functools

import jax
import jax.numpy as jnp
import numpy as np
from jax import lax
from jax.experimental import pallas as pl
from jax.experimental.pallas import tpu as pltpu
from jax.experimental.pallas import tpu_sc as plsc

_NC = 2
_NS = 16
_NW = _NC * _NS

LAT = 128
_EB = 8000
_NB = 2000
_F32 = jnp.float32


def _split_rows(W, dims):
    out, o = [], 0
    for d in dims:
        out.append(W[o:o + d])
        o += d
    return out


def _mm3_body(has_base, na, *refs):
    a = refs[:na]
    w = refs[na:2 * na]
    acc = jnp.dot(a[0][...], w[0][...], preferred_element_type=_F32)
    for i in range(1, na):
        acc = acc + jnp.dot(a[i][...], w[i][...], preferred_element_type=_F32)
    k = 2 * na
    for j in range(3):
        part = acc[:, j * LAT:(j + 1) * LAT]
        if has_base:
            part = part + refs[k + j][...]
        refs[-3 + j][...] = part


def _mm3(as_, ws, base3=None, block=_NB):
    R = as_[0].shape[0]
    K = ws[0].shape[1]
    na = len(as_)
    in_specs = (
        [pl.BlockSpec((block, a.shape[1]), lambda i: (i, 0)) for a in as_]
        + [pl.BlockSpec((w.shape[0], K), lambda i: (0, 0)) for w in ws]
    )
    args = list(as_) + list(ws)
    if base3 is not None:
        in_specs += [pl.BlockSpec((block, LAT), lambda i: (i, 0))] * 3
        args += list(base3)
    out = pl.pallas_call(
        functools.partial(_mm3_body, base3 is not None, na),
        grid=(R // block,),
        in_specs=in_specs,
        out_specs=[pl.BlockSpec((block, LAT), lambda i: (i, 0))] * 3,
        out_shape=[jax.ShapeDtypeStruct((R, LAT), _F32)] * 3,
    )(*args)
    return tuple(out)


def _edge_body(ne, *refs):
    acc = refs[0][...] + refs[1][...] + refs[2 + 2 * ne][...]
    for i in range(ne):
        acc = acc + jnp.dot(refs[2 + i][...], refs[2 + ne + i][...],
                            preferred_element_type=_F32)
    h = jnp.maximum(acc, 0.0)
    refs[-1][...] = (jnp.dot(h, refs[3 + 2 * ne][...],
                             preferred_element_type=_F32) + refs[4 + 2 * ne][...])


def _edge_mlp(gs, gd, ea_parts, we_parts, cvec, w2, b2):
    E = gs.shape[0]
    ne = len(ea_parts)
    d_out = w2.shape[1]
    in_specs = (
        [pl.BlockSpec((_EB, LAT), lambda i: (i, 0)),
         pl.BlockSpec((_EB, LAT), lambda i: (i, 0))]
        + [pl.BlockSpec((_EB, ea.shape[1]), lambda i: (i, 0)) for ea in ea_parts]
        + [pl.BlockSpec((we.shape[0], LAT), lambda i: (0, 0)) for we in we_parts]
        + [pl.BlockSpec((1, LAT), lambda i: (0, 0)),
           pl.BlockSpec((LAT, d_out), lambda i: (0, 0)),
           pl.BlockSpec((1, d_out), lambda i: (0, 0))]
    )
    return pl.pallas_call(
        functools.partial(_edge_body, ne),
        grid=(E // _EB,),
        in_specs=in_specs,
        out_specs=pl.BlockSpec((_EB, d_out), lambda i: (i, 0)),
        out_shape=jax.ShapeDtypeStruct((E, d_out), _F32),
    )(gs, gd, *ea_parts, *we_parts, cvec, w2, b2)


def _node_body(hx, aggp, va, cvec, v2, b2, out):
    agg = aggp[0] + aggp[1]
    h = jnp.maximum(hx[...] + jnp.dot(agg, va[...],
                                      preferred_element_type=_F32) + cvec[...], 0.0)
    out[...] = jnp.dot(h, v2[...], preferred_element_type=_F32) + b2[...]


def _node_mlp(hx, aggp, va, cvec, v2, b2):
    N = hx.shape[0]
    da = aggp.shape[2]
    d_out = v2.shape[1]
    return pl.pallas_call(
        _node_body,
        grid=(N // _NB,),
        in_specs=[
            pl.BlockSpec((_NB, LAT), lambda i: (i, 0)),
            pl.BlockSpec((_NC, _NB, da), lambda i: (0, i, 0)),
            pl.BlockSpec((da, LAT), lambda i: (0, 0)),
            pl.BlockSpec((1, LAT), lambda i: (0, 0)),
            pl.BlockSpec((LAT, d_out), lambda i: (0, 0)),
            pl.BlockSpec((1, d_out), lambda i: (0, 0)),
        ],
        out_specs=pl.BlockSpec((_NB, d_out), lambda i: (i, 0)),
        out_shape=jax.ShapeDtypeStruct((N, d_out), _F32),
    )(hx, aggp, va, cvec, v2, b2)


def _global_body(nu, inv_n, *refs):
    m = jnp.sum(refs[0][...], axis=0, keepdims=True) * inv_n
    acc = jnp.dot(m, refs[1 + nu][...], preferred_element_type=_F32)
    for i in range(nu):
        acc = acc + jnp.dot(refs[1 + i][...], refs[2 + nu + i][...],
                            preferred_element_type=_F32)
    h = jnp.maximum(acc + refs[2 + 2 * nu][...], 0.0)
    refs[-1][...] = (jnp.dot(h, refs[3 + 2 * nu][...],
                             preferred_element_type=_F32) + refs[4 + 2 * nu][...])


def _global_mlp(xn, u_parts, gm, gu_parts, b1, g2, b2):
    N = xn.shape[0]
    nu = len(u_parts)
    d_out = g2.shape[1]
    in_specs = (
        [pl.BlockSpec((N, LAT), lambda: (0, 0))]
        + [pl.BlockSpec((1, u.shape[1]), lambda: (0, 0)) for u in u_parts]
        + [pl.BlockSpec((LAT, LAT), lambda: (0, 0))]
        + [pl.BlockSpec((w.shape[0], LAT), lambda: (0, 0)) for w in gu_parts]
        + [pl.BlockSpec((1, LAT), lambda: (0, 0)),
           pl.BlockSpec((LAT, d_out), lambda: (0, 0)),
           pl.BlockSpec((1, d_out), lambda: (0, 0))]
    )
    return pl.pallas_call(
        functools.partial(_global_body, nu, 1.0 / N),
        in_specs=in_specs,
        out_specs=pl.BlockSpec((1, d_out), lambda: (0, 0)),
        out_shape=jax.ShapeDtypeStruct((1, d_out), _F32),
    )(xn, *u_parts, gm, *gu_parts, b1, g2, b2)


def _cvec_body(nu, *refs):
    acc_e = refs[3 * nu][...]
    acc_n = refs[3 * nu + 1][...]
    for i in range(nu):
        u = refs[i][...]
        acc_e = acc_e + jnp.dot(u, refs[nu + i][...], preferred_element_type=_F32)
        acc_n = acc_n + jnp.dot(u, refs[2 * nu + i][...],
                                preferred_element_type=_F32)
    refs[-2][...] = acc_e
    refs[-1][...] = acc_n


_GWG = 200
_GWS = 2000


def _gather_body(nwin, xs_hbm, xd_hbm, idx_hbm, gs_hbm, gd_hbm,
                 ix0, ix1, a0, b0, a1, b1, si0, si1, sg0, sg1, sw0, sw1):
    wid = lax.axis_index("s") * _NC + lax.axis_index("c")
    base = wid * nwin
    ix = (ix0, ix1)
    ab = ((a0, b0), (a1, b1))
    si = (si0, si1)
    sg = (sg0, sg1)
    sw = (sw0, sw1)

    def prefetch(w, slot):
        pltpu.async_copy(idx_hbm.at[base + w], ix[slot], si[slot])

    def window(w, slot, first):
        a, b = ab[slot]
        if not first:
            pltpu.make_async_copy(a, gs_hbm.at[pl.ds(0, _GWG)], sw[slot]).wait()
            pltpu.make_async_copy(b, gd_hbm.at[pl.ds(0, _GWG)], sw[slot]).wait()
        pltpu.make_async_copy(idx_hbm.at[base], ix[slot], si[slot]).wait()
        cpa = pltpu.async_copy(xs_hbm.at[ix[slot].at[pl.ds(0, _GWG)]], a, sg[slot])
        cpb = pltpu.async_copy(xd_hbm.at[ix[slot].at[pl.ds(_GWG, _GWG)]], b, sg[slot])
        cpa.wait()
        cpb.wait()
        if isinstance(w, int):
            if w + 2 < nwin:
                prefetch(w + 2, slot)
        else:
            @pl.when(w + 2 < nwin)
            def _():
                prefetch(w + 2, slot)
        pltpu.async_copy(a, gs_hbm.at[pl.ds((base + w) * _GWG, _GWG)], sw[slot])
        pltpu.async_copy(b, gd_hbm.at[pl.ds((base + w) * _GWG, _GWG)], sw[slot])

    prefetch(0, 0)
    prefetch(1, 1)
    window(0, 0, True)
    window(1, 1, True)

    def step(k, carry):
        window(2 * k, 0, False)
        window(2 * k + 1, 1, False)
        return carry

    lax.fori_loop(1, nwin // 2, step, 0)
    for slot in (0, 1):
        pltpu.make_async_copy(ab[slot][0], gs_hbm.at[pl.ds(0, _GWG)], sw[slot]).wait()
        pltpu.make_async_copy(ab[slot][1], gd_hbm.at[pl.ds(0, _GWG)], sw[slot]).wait()


def _sc_gather2(xs, xd, idx2, E):
    nwin = idx2.shape[0] // _NW
    mesh = plsc.VectorSubcoreMesh(core_axis_name="c", subcore_axis_name="s",
                                  num_cores=_NC, num_subcores=_NS)
    fn = pl.kernel(
        functools.partial(_gather_body, nwin),
        out_type=[jax.ShapeDtypeStruct((E, LAT), _F32),
                  jax.ShapeDtypeStruct((E, LAT), _F32)],
        mesh=mesh,
        scratch_types=[
            pltpu.VMEM((2 * _GWG,), jnp.int32),
            pltpu.VMEM((2 * _GWG,), jnp.int32),
            pltpu.VMEM((_GWG, LAT), _F32),
            pltpu.VMEM((_GWG, LAT), _F32),
            pltpu.VMEM((_GWG, LAT), _F32),
            pltpu.VMEM((_GWG, LAT), _F32),
            pltpu.SemaphoreType.DMA,
            pltpu.SemaphoreType.DMA,
            pltpu.SemaphoreType.DMA,
            pltpu.SemaphoreType.DMA,
            pltpu.SemaphoreType.DMA,
            pltpu.SemaphoreType.DMA,
        ],
    )
    return fn(xs, xd, idx2)


def _scatter_body(nwin, nchunk, e_hbm, cidx_hbm, zero_hbm, out_hbm,
                  idx_v, upd_v, obuf, acc):
    cid = lax.axis_index("c")
    sid = lax.axis_index("s")
    wid = sid * _NC + cid
    nrows = acc.shape[0]

    @pl.when(sid * nchunk < nrows)
    def _():
        pltpu.sync_copy(zero_hbm.at[pl.ds(sid * nchunk, nchunk)], obuf)
        pltpu.sync_copy(obuf, acc.at[pl.ds(sid * nchunk, nchunk)])

    plsc.subcore_barrier()

    def step(j, carry):
        r = wid * nwin + j
        pltpu.sync_copy(cidx_hbm.at[r], idx_v)
        pltpu.sync_copy(e_hbm.at[pl.ds(r * _GWS, _GWS)], upd_v)
        pltpu.sync_copy(upd_v, acc.at[idx_v], add=True)
        return carry

    lax.fori_loop(0, nwin, step, 0)
    plsc.subcore_barrier()

    @pl.when(sid * nchunk < nrows)
    def _():
        pltpu.sync_copy(acc.at[pl.ds(sid * nchunk, nchunk)], obuf)
        pltpu.sync_copy(obuf, out_hbm.at[cid, pl.ds(sid * nchunk, nchunk)])


def _sc_scatter_add(e, cidx2, N):
    E, de = e.shape
    nwin = cidx2.shape[0] // _NW
    nchunk = 1000
    mesh = plsc.VectorSubcoreMesh(core_axis_name="c", subcore_axis_name="s",
                                  num_cores=_NC, num_subcores=_NS)
    fn = pl.kernel(
        functools.partial(_scatter_body, nwin, nchunk),
        out_type=jax.ShapeDtypeStruct((_NC, N, de), _F32),
        mesh=mesh,
        compiler_params=pltpu.CompilerParams(use_tc_tiling_on_sc=False),
        scratch_types=[
            pltpu.VMEM((_GWS,), jnp.int32),
            pltpu.VMEM((_GWS, de), _F32),
            pltpu.VMEM((1000, de), _F32),
            pltpu.VMEM_SHARED((N, de), _F32),
        ],
    )
    return fn(e, cidx2, jnp.zeros((N, de), _F32))


def _weight_views(p, dims_x, dims_e, dims_u, de_out):
    pe, pn, pg = p["edge"], p["node"], p["global"]
    nx, nee, nuu = len(dims_x), len(dims_e), len(dims_u)
    parts = _split_rows(pe["w1"], dims_x + dims_x + dims_e + dims_u)
    W_src = parts[:nx]
    W_dst = parts[nx:2 * nx]
    W_ea = parts[2 * nx:2 * nx + nee]
    W_eu = parts[2 * nx + nee:]
    parts = _split_rows(pn["w1"], dims_x + [de_out] + dims_u)
    V_x = parts[:nx]
    V_a = parts[nx]
    V_u = parts[nx + 1:]
    parts = _split_rows(pg["w1"], [LAT] + dims_u)
    G_m = parts[0]
    G_u = parts[1:]
    W_prep = [jnp.concatenate([W_src[i], W_dst[i], V_x[i]], axis=1)
              for i in range(nx)]
    return dict(W_prep=W_prep, W_ea=W_ea, W_eu=W_eu, V_a=V_a, V_u=V_u,
                G_m=G_m, G_u=G_u, pe=pe, pn=pn, pg=pg)


def _apply(wv, x_parts, ea_parts, u_parts, idx2, cidx2s, N, E, prep_base=None):
    if prep_base is None:
        xs, xd, hx = _mm3(x_parts, wv["W_prep"])
    else:
        xs, xd, hx = _mm3([x_parts[-1]], [wv["W_prep"][-1]], base3=prep_base)
    cvec_e, cvec_n = _cvecs_call(u_parts, wv["W_eu"], wv["V_u"],
                                 wv["pe"]["b1"].reshape(1, LAT),
                                 wv["pn"]["b1"].reshape(1, LAT))
    gs, gd = _sc_gather2(xs, xd, idx2, E)
    e = _edge_mlp(gs, gd, ea_parts, wv["W_ea"], cvec_e,
                  wv["pe"]["w2"], wv["pe"]["b2"].reshape(1, -1))
    aggp = _sc_scatter_add(e, cidx2s, N)
    xn = _node_mlp(hx, aggp, wv["V_a"], cvec_n,
                   wv["pn"]["w2"], wv["pn"]["b2"].reshape(1, -1))
    gu = _global_mlp(xn, u_parts, wv["G_m"], wv["G_u"],
                     wv["pg"]["b1"].reshape(1, LAT),
                     wv["pg"]["w2"], wv["pg"]["b2"].reshape(1, -1))
    return xn, e, gu


def _cvecs_call(u_parts, we_parts, wn_parts, b1e, b1n):
    nu = len(u_parts)
    return pl.pallas_call(
        functools.partial(_cvec_body, nu),
        in_specs=(
            [pl.BlockSpec((1, u.shape[1]), lambda: (0, 0)) for u in u_parts]
            + [pl.BlockSpec((w.shape[0], LAT), lambda: (0, 0)) for w in we_parts]
            + [pl.BlockSpec((w.shape[0], LAT), lambda: (0, 0)) for w in wn_parts]
            + [pl.BlockSpec((1, LAT), lambda: (0, 0)),
               pl.BlockSpec((1, LAT), lambda: (0, 0))]
        ),
        out_specs=[pl.BlockSpec((1, LAT), lambda: (0, 0)),
                   pl.BlockSpec((1, LAT), lambda: (0, 0))],
        out_shape=[jax.ShapeDtypeStruct((1, LAT), _F32),
                   jax.ShapeDtypeStruct((1, LAT), _F32)],
    )(*u_parts, *we_parts, *wn_parts, b1e, b1n)


def kernel(x, edge_attr, global_attr, params, edge_index):
    row, col = edge_index[0], edge_index[1]
    N = x.shape[0]
    E = edge_attr.shape[0]
    DN, DE, DG = x.shape[1], edge_attr.shape[1], global_attr.shape[1]

    wv_enc = _weight_views(params["encoder"], [DN], [DE], [DG], DE)
    wv_core = _weight_views(params["core"], [DN, DN], [DE, DE], [DG, DG], DE)
    wv_dec = _weight_views(params["decoder"], [DN], [DE], [DG], DE)

    idx2 = jnp.concatenate([row.reshape(-1, _GWG), col.reshape(-1, _GWG)],
                           axis=1)
    cidx2s = col.reshape(-1, _GWS)

    x1, e1, u1 = _apply(wv_enc, [x], [edge_attr], [global_attr],
                        idx2, cidx2s, N, E)
    x0, e0, u0 = x1, e1, u1

    prep_base0 = _mm3([x0], [wv_core["W_prep"][0]])
    xc, ec, uc = x1, e1, u1
    for _ in range(5):
        xc, ec, uc = _apply(wv_core, [x0, xc], [e0, ec], [u0, uc],
                            idx2, cidx2s, N, E, prep_base=prep_base0)

    return _apply(wv_dec, [xc], [ec], [uc], idx2, cidx2s, N, E)


# speedup vs baseline: 4.5320x; 1.0028x over previous
import functools

import jax
import jax.numpy as jnp
import numpy as np
from jax import lax
from jax.experimental import pallas as pl
from jax.experimental.pallas import tpu as pltpu
from jax.experimental.pallas import tpu_sc as plsc

_NC = 2
_NS = 16
_NW = _NC * _NS

LAT = 128
_EB = 8000
_NB = 2000
_F32 = jnp.float32


def _split_rows(W, dims):
    out, o = [], 0
    for d in dims:
        out.append(W[o:o + d])
        o += d
    return out


def _mm3_body(has_base, na, *refs):
    a = refs[:na]
    w = refs[na:2 * na]
    acc = jnp.dot(a[0][...], w[0][...], preferred_element_type=_F32)
    for i in range(1, na):
        acc = acc + jnp.dot(a[i][...], w[i][...], preferred_element_type=_F32)
    k = 2 * na
    for j in range(3):
        part = acc[:, j * LAT:(j + 1) * LAT]
        if has_base:
            part = part + refs[k + j][...]
        refs[-3 + j][...] = part


def _mm3(as_, ws, base3=None, block=_NB):
    R = as_[0].shape[0]
    K = ws[0].shape[1]
    na = len(as_)
    in_specs = (
        [pl.BlockSpec((block, a.shape[1]), lambda i: (i, 0)) for a in as_]
        + [pl.BlockSpec((w.shape[0], K), lambda i: (0, 0)) for w in ws]
    )
    args = list(as_) + list(ws)
    if base3 is not None:
        in_specs += [pl.BlockSpec((block, LAT), lambda i: (i, 0))] * 3
        args += list(base3)
    out = pl.pallas_call(
        functools.partial(_mm3_body, base3 is not None, na),
        grid=(R // block,),
        in_specs=in_specs,
        out_specs=[pl.BlockSpec((block, LAT), lambda i: (i, 0))] * 3,
        out_shape=[jax.ShapeDtypeStruct((R, LAT), _F32)] * 3,
    )(*args)
    return tuple(out)


def _edge_body(ne, *refs):
    acc = refs[0][...] + refs[1][...] + refs[2 + 2 * ne][...]
    for i in range(ne):
        acc = acc + jnp.dot(refs[2 + i][...], refs[2 + ne + i][...],
                            preferred_element_type=_F32)
    h = jnp.maximum(acc, 0.0)
    refs[-1][...] = (jnp.dot(h, refs[3 + 2 * ne][...],
                             preferred_element_type=_F32) + refs[4 + 2 * ne][...])


def _edge_mlp(gs, gd, ea_parts, we_parts, cvec, w2, b2):
    E = gs.shape[0]
    ne = len(ea_parts)
    d_out = w2.shape[1]
    in_specs = (
        [pl.BlockSpec((_EB, LAT), lambda i: (i, 0)),
         pl.BlockSpec((_EB, LAT), lambda i: (i, 0))]
        + [pl.BlockSpec((_EB, ea.shape[1]), lambda i: (i, 0)) for ea in ea_parts]
        + [pl.BlockSpec((we.shape[0], LAT), lambda i: (0, 0)) for we in we_parts]
        + [pl.BlockSpec((1, LAT), lambda i: (0, 0)),
           pl.BlockSpec((LAT, d_out), lambda i: (0, 0)),
           pl.BlockSpec((1, d_out), lambda i: (0, 0))]
    )
    return pl.pallas_call(
        functools.partial(_edge_body, ne),
        grid=(E // _EB,),
        in_specs=in_specs,
        out_specs=pl.BlockSpec((_EB, d_out), lambda i: (i, 0)),
        out_shape=jax.ShapeDtypeStruct((E, d_out), _F32),
    )(gs, gd, *ea_parts, *we_parts, cvec, w2, b2)


def _node_body(hx, aggp, va, cvec, v2, b2, out):
    agg = aggp[0] + aggp[1]
    h = jnp.maximum(hx[...] + jnp.dot(agg, va[...],
                                      preferred_element_type=_F32) + cvec[...], 0.0)
    out[...] = jnp.dot(h, v2[...], preferred_element_type=_F32) + b2[...]


def _node_mlp(hx, aggp, va, cvec, v2, b2):
    N = hx.shape[0]
    da = aggp.shape[2]
    d_out = v2.shape[1]
    return pl.pallas_call(
        _node_body,
        grid=(N // _NB,),
        in_specs=[
            pl.BlockSpec((_NB, LAT), lambda i: (i, 0)),
            pl.BlockSpec((_NC, _NB, da), lambda i: (0, i, 0)),
            pl.BlockSpec((da, LAT), lambda i: (0, 0)),
            pl.BlockSpec((1, LAT), lambda i: (0, 0)),
            pl.BlockSpec((LAT, d_out), lambda i: (0, 0)),
            pl.BlockSpec((1, d_out), lambda i: (0, 0)),
        ],
        out_specs=pl.BlockSpec((_NB, d_out), lambda i: (i, 0)),
        out_shape=jax.ShapeDtypeStruct((N, d_out), _F32),
    )(hx, aggp, va, cvec, v2, b2)


def _global_body(nu, inv_n, *refs):
    m = jnp.sum(refs[0][...], axis=0, keepdims=True) * inv_n
    acc = jnp.dot(m, refs[1 + nu][...], preferred_element_type=_F32)
    for i in range(nu):
        acc = acc + jnp.dot(refs[1 + i][...], refs[2 + nu + i][...],
                            preferred_element_type=_F32)
    h = jnp.maximum(acc + refs[2 + 2 * nu][...], 0.0)
    refs[-1][...] = (jnp.dot(h, refs[3 + 2 * nu][...],
                             preferred_element_type=_F32) + refs[4 + 2 * nu][...])


def _global_mlp(xn, u_parts, gm, gu_parts, b1, g2, b2):
    N = xn.shape[0]
    nu = len(u_parts)
    d_out = g2.shape[1]
    in_specs = (
        [pl.BlockSpec((N, LAT), lambda: (0, 0))]
        + [pl.BlockSpec((1, u.shape[1]), lambda: (0, 0)) for u in u_parts]
        + [pl.BlockSpec((LAT, LAT), lambda: (0, 0))]
        + [pl.BlockSpec((w.shape[0], LAT), lambda: (0, 0)) for w in gu_parts]
        + [pl.BlockSpec((1, LAT), lambda: (0, 0)),
           pl.BlockSpec((LAT, d_out), lambda: (0, 0)),
           pl.BlockSpec((1, d_out), lambda: (0, 0))]
    )
    return pl.pallas_call(
        functools.partial(_global_body, nu, 1.0 / N),
        in_specs=in_specs,
        out_specs=pl.BlockSpec((1, d_out), lambda: (0, 0)),
        out_shape=jax.ShapeDtypeStruct((1, d_out), _F32),
    )(xn, *u_parts, gm, *gu_parts, b1, g2, b2)


def _cvec_body(nu, *refs):
    acc_e = refs[3 * nu][...]
    acc_n = refs[3 * nu + 1][...]
    for i in range(nu):
        u = refs[i][...]
        acc_e = acc_e + jnp.dot(u, refs[nu + i][...], preferred_element_type=_F32)
        acc_n = acc_n + jnp.dot(u, refs[2 * nu + i][...],
                                preferred_element_type=_F32)
    refs[-2][...] = acc_e
    refs[-1][...] = acc_n


_GWG = 200
_GWS = 2000


def _gather_body(nwin, xs_hbm, xd_hbm, idx_hbm, gs_hbm, gd_hbm,
                 ix0, ix1, a0, b0, a1, b1, si0, si1, sg0, sg1, sw0, sw1):
    wid = lax.axis_index("s") * _NC + lax.axis_index("c")
    base = wid * nwin
    ix = (ix0, ix1)
    ab = ((a0, b0), (a1, b1))
    si = (si0, si1)
    sg = (sg0, sg1)
    sw = (sw0, sw1)

    def prefetch(w, slot):
        pltpu.async_copy(idx_hbm.at[base + w], ix[slot], si[slot])

    def wait_idx(slot):
        pltpu.make_async_copy(idx_hbm.at[base], ix[slot], si[slot]).wait()

    def issue_gathers(slot):
        a, b = ab[slot]
        pltpu.async_copy(xs_hbm.at[ix[slot].at[pl.ds(0, _GWG)]], a, sg[slot])
        pltpu.async_copy(xd_hbm.at[ix[slot].at[pl.ds(_GWG, _GWG)]], b, sg[slot])

    def wait_gathers(slot):
        a, b = ab[slot]
        pltpu.make_async_copy(a, gs_hbm.at[pl.ds(0, _GWG)], sg[slot]).wait()
        pltpu.make_async_copy(b, gd_hbm.at[pl.ds(0, _GWG)], sg[slot]).wait()

    def write_out(w, slot):
        a, b = ab[slot]
        pltpu.async_copy(a, gs_hbm.at[pl.ds((base + w) * _GWG, _GWG)], sw[slot])
        pltpu.async_copy(b, gd_hbm.at[pl.ds((base + w) * _GWG, _GWG)], sw[slot])

    def wait_writes(slot):
        a, b = ab[slot]
        pltpu.make_async_copy(a, gs_hbm.at[pl.ds(0, _GWG)], sw[slot]).wait()
        pltpu.make_async_copy(b, gd_hbm.at[pl.ds(0, _GWG)], sw[slot]).wait()

    def halfstep(w, s, first):
        o = 1 - s
        if not first:
            wait_writes(s)
        wait_idx(s)
        issue_gathers(s)
        wait_gathers(o)
        if isinstance(w, int):
            if w + 1 < nwin:
                prefetch(w + 1, o)
        else:
            @pl.when(w + 1 < nwin)
            def _():
                prefetch(w + 1, o)
        write_out(w - 1, o)

    prefetch(0, 0)
    prefetch(1, 1)
    wait_idx(0)
    issue_gathers(0)
    halfstep(1, 1, True)

    def step(k, carry):
        halfstep(2 * k, 0, False)
        halfstep(2 * k + 1, 1, False)
        return carry

    lax.fori_loop(1, nwin // 2, step, 0)
    wait_gathers(1)
    write_out(nwin - 1, 1)
    wait_writes(0)
    wait_writes(1)


def _sc_gather2(xs, xd, idx2, E):
    nwin = idx2.shape[0] // _NW
    mesh = plsc.VectorSubcoreMesh(core_axis_name="c", subcore_axis_name="s",
                                  num_cores=_NC, num_subcores=_NS)
    fn = pl.kernel(
        functools.partial(_gather_body, nwin),
        out_type=[jax.ShapeDtypeStruct((E, LAT), _F32),
                  jax.ShapeDtypeStruct((E, LAT), _F32)],
        mesh=mesh,
        scratch_types=[
            pltpu.VMEM((2 * _GWG,), jnp.int32),
            pltpu.VMEM((2 * _GWG,), jnp.int32),
            pltpu.VMEM((_GWG, LAT), _F32),
            pltpu.VMEM((_GWG, LAT), _F32),
            pltpu.VMEM((_GWG, LAT), _F32),
            pltpu.VMEM((_GWG, LAT), _F32),
            pltpu.SemaphoreType.DMA,
            pltpu.SemaphoreType.DMA,
            pltpu.SemaphoreType.DMA,
            pltpu.SemaphoreType.DMA,
            pltpu.SemaphoreType.DMA,
            pltpu.SemaphoreType.DMA,
        ],
    )
    return fn(xs, xd, idx2)


def _scatter_body(nwin, nchunk, e_hbm, cidx_hbm, zero_hbm, out_hbm,
                  idx_v, upd_v, obuf, acc):
    cid = lax.axis_index("c")
    sid = lax.axis_index("s")
    wid = sid * _NC + cid
    nrows = acc.shape[0]

    @pl.when(sid * nchunk < nrows)
    def _():
        pltpu.sync_copy(zero_hbm.at[pl.ds(sid * nchunk, nchunk)], obuf)
        pltpu.sync_copy(obuf, acc.at[pl.ds(sid * nchunk, nchunk)])

    plsc.subcore_barrier()

    def step(j, carry):
        r = wid * nwin + j
        pltpu.sync_copy(cidx_hbm.at[r], idx_v)
        pltpu.sync_copy(e_hbm.at[pl.ds(r * _GWS, _GWS)], upd_v)
        pltpu.sync_copy(upd_v, acc.at[idx_v], add=True)
        return carry

    lax.fori_loop(0, nwin, step, 0)
    plsc.subcore_barrier()

    @pl.when(sid * nchunk < nrows)
    def _():
        pltpu.sync_copy(acc.at[pl.ds(sid * nchunk, nchunk)], obuf)
        pltpu.sync_copy(obuf, out_hbm.at[cid, pl.ds(sid * nchunk, nchunk)])


def _sc_scatter_add(e, cidx2, N):
    E, de = e.shape
    nwin = cidx2.shape[0] // _NW
    nchunk = 1000
    mesh = plsc.VectorSubcoreMesh(core_axis_name="c", subcore_axis_name="s",
                                  num_cores=_NC, num_subcores=_NS)
    fn = pl.kernel(
        functools.partial(_scatter_body, nwin, nchunk),
        out_type=jax.ShapeDtypeStruct((_NC, N, de), _F32),
        mesh=mesh,
        compiler_params=pltpu.CompilerParams(use_tc_tiling_on_sc=False),
        scratch_types=[
            pltpu.VMEM((_GWS,), jnp.int32),
            pltpu.VMEM((_GWS, de), _F32),
            pltpu.VMEM((1000, de), _F32),
            pltpu.VMEM_SHARED((N, de), _F32),
        ],
    )
    return fn(e, cidx2, jnp.zeros((N, de), _F32))


def _weight_views(p, dims_x, dims_e, dims_u, de_out):
    pe, pn, pg = p["edge"], p["node"], p["global"]
    nx, nee, nuu = len(dims_x), len(dims_e), len(dims_u)
    parts = _split_rows(pe["w1"], dims_x + dims_x + dims_e + dims_u)
    W_src = parts[:nx]
    W_dst = parts[nx:2 * nx]
    W_ea = parts[2 * nx:2 * nx + nee]
    W_eu = parts[2 * nx + nee:]
    parts = _split_rows(pn["w1"], dims_x + [de_out] + dims_u)
    V_x = parts[:nx]
    V_a = parts[nx]
    V_u = parts[nx + 1:]
    parts = _split_rows(pg["w1"], [LAT] + dims_u)
    G_m = parts[0]
    G_u = parts[1:]
    W_prep = [jnp.concatenate([W_src[i], W_dst[i], V_x[i]], axis=1)
              for i in range(nx)]
    return dict(W_prep=W_prep, W_ea=W_ea, W_eu=W_eu, V_a=V_a, V_u=V_u,
                G_m=G_m, G_u=G_u, pe=pe, pn=pn, pg=pg)


def _apply(wv, x_parts, ea_parts, u_parts, idx2, cidx2s, N, E, prep_base=None):
    if prep_base is None:
        xs, xd, hx = _mm3(x_parts, wv["W_prep"])
    else:
        xs, xd, hx = _mm3([x_parts[-1]], [wv["W_prep"][-1]], base3=prep_base)
    cvec_e, cvec_n = _cvecs_call(u_parts, wv["W_eu"], wv["V_u"],
                                 wv["pe"]["b1"].reshape(1, LAT),
                                 wv["pn"]["b1"].reshape(1, LAT))
    gs, gd = _sc_gather2(xs, xd, idx2, E)
    e = _edge_mlp(gs, gd, ea_parts, wv["W_ea"], cvec_e,
                  wv["pe"]["w2"], wv["pe"]["b2"].reshape(1, -1))
    aggp = _sc_scatter_add(e, cidx2s, N)
    xn = _node_mlp(hx, aggp, wv["V_a"], cvec_n,
                   wv["pn"]["w2"], wv["pn"]["b2"].reshape(1, -1))
    gu = _global_mlp(xn, u_parts, wv["G_m"], wv["G_u"],
                     wv["pg"]["b1"].reshape(1, LAT),
                     wv["pg"]["w2"], wv["pg"]["b2"].reshape(1, -1))
    return xn, e, gu


def _cvecs_call(u_parts, we_parts, wn_parts, b1e, b1n):
    nu = len(u_parts)
    return pl.pallas_call(
        functools.partial(_cvec_body, nu),
        in_specs=(
            [pl.BlockSpec((1, u.shape[1]), lambda: (0, 0)) for u in u_parts]
            + [pl.BlockSpec((w.shape[0], LAT), lambda: (0, 0)) for w in we_parts]
            + [pl.BlockSpec((w.shape[0], LAT), lambda: (0, 0)) for w in wn_parts]
            + [pl.BlockSpec((1, LAT), lambda: (0, 0)),
               pl.BlockSpec((1, LAT), lambda: (0, 0))]
        ),
        out_specs=[pl.BlockSpec((1, LAT), lambda: (0, 0)),
                   pl.BlockSpec((1, LAT), lambda: (0, 0))],
        out_shape=[jax.ShapeDtypeStruct((1, LAT), _F32),
                   jax.ShapeDtypeStruct((1, LAT), _F32)],
    )(*u_parts, *we_parts, *wn_parts, b1e, b1n)


def kernel(x, edge_attr, global_attr, params, edge_index):
    row, col = edge_index[0], edge_index[1]
    N = x.shape[0]
    E = edge_attr.shape[0]
    DN, DE, DG = x.shape[1], edge_attr.shape[1], global_attr.shape[1]

    wv_enc = _weight_views(params["encoder"], [DN], [DE], [DG], DE)
    wv_core = _weight_views(params["core"], [DN, DN], [DE, DE], [DG, DG], DE)
    wv_dec = _weight_views(params["decoder"], [DN], [DE], [DG], DE)

    idx2 = jnp.concatenate([row.reshape(-1, _GWG), col.reshape(-1, _GWG)],
                           axis=1)
    cidx2s = col.reshape(-1, _GWS)

    x1, e1, u1 = _apply(wv_enc, [x], [edge_attr], [global_attr],
                        idx2, cidx2s, N, E)
    x0, e0, u0 = x1, e1, u1

    prep_base0 = _mm3([x0], [wv_core["W_prep"][0]])
    xc, ec, uc = x1, e1, u1
    for _ in range(5):
        xc, ec, uc = _apply(wv_core, [x0, xc], [e0, ec], [u0, uc],
                            idx2, cidx2s, N, E, prep_base=prep_base0)

    return _apply(wv_dec, [xc], [ec], [uc], idx2, cidx2s, N, E)
